# baseline (speedup 1.0000x reference)
import functools

import jax
import jax.numpy as jnp
import numpy as np
from jax import lax
from jax.experimental import pallas as pl
from jax.experimental.pallas import tpu as pltpu
from jax.experimental.pallas import tpu_sc as plsc

N = 10000
E = 320000
HEADS = 8
OPH = 16
LAYERS = 2
HOPS = 2
DECAY = [float(np.exp(-0.5 * k)) for k in range(HOPS)]

NC, NS, LANES = 2, 16, 16
NW = NC * NS
NP = 10240
NSL = NP // NS
TE = E // NW

_MESH = plsc.VectorSubcoreMesh(
    core_axis_name="c", subcore_axis_name="s", num_cores=NC, num_subcores=NS)

_IOTA16 = lambda: lax.iota(jnp.int32, 16)


def _vgather(v, idx):
    return lax.gather(
        v, idx[:, None],
        lax.GatherDimensionNumbers(
            offset_dims=(), collapsed_slice_dims=(0,), start_index_map=(0,)),
        (1,), mode=lax.GatherScatterMode.PROMISE_IN_BOUNDS)


def _nbr_body(src_hbm, dst_hbm, out_hbm, src_v, dst_v, tbl_v, blk_v, acc_v, shr):
    c = lax.axis_index("c")
    s = lax.axis_index("s")
    wid = s * NC + c
    pltpu.sync_copy(src_hbm.at[pl.ds(wid * TE, TE)], src_v)
    pltpu.sync_copy(dst_hbm.at[pl.ds(wid * TE, TE)], dst_v)

    def zbody(i, _):
        tbl_v[pl.ds(i * 16, 16)] = jnp.zeros((16,), jnp.int32)
        return 0
    lax.fori_loop(0, NP // 16, zbody, 0)

    iot = _IOTA16()

    def ebody(i, _):
        sv = src_v[pl.ds(i * 16, 16)]
        dv = dst_v[pl.ds(i * 16, 16)]
        ks, _ = plsc.sort_key_val(sv * 16384 + dv, dv)
        ss = lax.shift_right_logical(ks, 14)
        dd = jnp.bitwise_and(ks, 16383)
        nxt = _vgather(ss, jnp.minimum(iot + 1, 15))
        is_end = jnp.logical_or(ss != nxt, iot == 15)
        old = plsc.load_gather(tbl_v, [ss], mask=is_end)
        plsc.store_scatter(tbl_v, [ss], jnp.maximum(old, dd), mask=is_end)
        return 0
    lax.fori_loop(0, TE // 16, ebody, 0)

    pltpu.sync_copy(tbl_v, shr.at[s])
    plsc.subcore_barrier()
    for r in range(NS):
        pltpu.sync_copy(shr.at[r, pl.ds(s * NSL, NSL)],
                        blk_v.at[pl.ds(r * NSL, NSL)])

    def cbody(j, _):
        m = blk_v[pl.ds(j * 16, 16)]
        for r in range(1, NS):
            m = jnp.maximum(m, blk_v[pl.ds(r * NSL + j * 16, 16)])
        acc_v[pl.ds(j * 16, 16)] = m
        return 0
    lax.fori_loop(0, NSL // 16, cbody, 0)
    pltpu.sync_copy(acc_v, out_hbm.at[c, pl.ds(s * NSL, NSL)])


@jax.jit
def _nbr_partials(src, dst):
    return pl.kernel(
        _nbr_body,
        out_type=jax.ShapeDtypeStruct((NC, NP), jnp.int32),
        mesh=_MESH,
        compiler_params=pltpu.CompilerParams(needs_layout_passes=False),
        scratch_types=[
            pltpu.VMEM((TE,), jnp.int32),
            pltpu.VMEM((TE,), jnp.int32),
            pltpu.VMEM((NP,), jnp.int32),
            pltpu.VMEM((NS * NSL,), jnp.int32),
            pltpu.VMEM((NSL,), jnp.int32),
            pltpu.VMEM_SHARED((NS, NP), jnp.int32),
        ],
    )(src, dst)


def _dst1_body(dst_hbm, nbr_hbm, out_hbm, dst_v, t0_v, t1_v, o_v):
    c = lax.axis_index("c")
    s = lax.axis_index("s")
    wid = s * NC + c
    pltpu.sync_copy(dst_hbm.at[pl.ds(wid * TE, TE)], dst_v)
    pltpu.sync_copy(nbr_hbm.at[0], t0_v)
    pltpu.sync_copy(nbr_hbm.at[1], t1_v)

    def mb(j, _):
        t0_v[pl.ds(j * 16, 16)] = jnp.maximum(
            t0_v[pl.ds(j * 16, 16)], t1_v[pl.ds(j * 16, 16)])
        return 0
    lax.fori_loop(0, NP // 16, mb, 0)

    def eb(i, _):
        dv = dst_v[pl.ds(i * 16, 16)]
        o_v[pl.ds(i * 16, 16)] = plsc.load_gather(t0_v, [dv])
        return 0
    lax.fori_loop(0, TE // 16, eb, 0)
    pltpu.sync_copy(o_v, out_hbm.at[pl.ds(wid * TE, TE)])


@jax.jit
def _dst1_compute(dst, nbr_p):
    return pl.kernel(
        _dst1_body,
        out_type=jax.ShapeDtypeStruct((E,), jnp.int32),
        mesh=_MESH,
        compiler_params=pltpu.CompilerParams(needs_layout_passes=False),
        scratch_types=[
            pltpu.VMEM((TE,), jnp.int32),
            pltpu.VMEM((NP,), jnp.int32),
            pltpu.VMEM((NP,), jnp.int32),
            pltpu.VMEM((TE,), jnp.int32),
        ],
    )(dst, nbr_p)


def _lrelu(v, s):
    return jnp.where(v >= 0, v, s * v)


def _stage0_body(x_ref, w_ref, b_ref, o_ref):
    h = jnp.dot(x_ref[...], w_ref[...], preferred_element_type=jnp.float32) + b_ref[...]
    o_ref[...] = _lrelu(h, 0.01)


def _stage0(x, W1, b1):
    B = 400
    return pl.pallas_call(
        _stage0_body,
        grid=(N // B,),
        in_specs=[
            pl.BlockSpec((B, 128), lambda i: (i, 0)),
            pl.BlockSpec((128, 128), lambda i: (0, 0)),
            pl.BlockSpec((1, 128), lambda i: (0, 0)),
        ],
        out_specs=pl.BlockSpec((B, 128), lambda i: (i, 0)),
        out_shape=jax.ShapeDtypeStruct((N, 128), jnp.float32),
    )(x, W1, b1.reshape(1, 128))


E2 = E + N
EPAD = 330240
TEP = EPAD // NW
CA = 1032


def _pass_a_body(srcp_hbm, dst0_hbm, dst1_hbm, hsd0_hbm, hsd1_hbm, z8_hbm,
                 ex0_hbm, ex1_hbm, den_hbm,
                 src_idx, dst_idx, rows_s, rows_d, ex_buf,
                 den_sp0, den_sp1, sem0, sem1):
    c = lax.axis_index("c")
    s = lax.axis_index("s")
    wid = s * NC + c
    iot = _IOTA16()

    pltpu.sync_copy(z8_hbm.at[pl.ds(s * NSL, NSL)], den_sp0.at[pl.ds(s * NSL, NSL)])
    pltpu.sync_copy(z8_hbm.at[pl.ds(s * NSL, NSL)], den_sp1.at[pl.ds(s * NSL, NSL)])
    pltpu.sync_copy(z8_hbm.at[pl.ds(0, CA), :], ex_buf)
    plsc.subcore_barrier()

    def row16(ref, r):
        return plsc.load_gather(ref, [jnp.full((16,), r, jnp.int32), iot])

    for k in range(HOPS):
        dst_hbm = dst0_hbm if k == 0 else dst1_hbm
        hsd_hbm = hsd0_hbm if k == 0 else hsd1_hbm
        ex_hbm = ex0_hbm if k == 0 else ex1_hbm
        den_sp = den_sp0 if k == 0 else den_sp1

        def chunk_body(cb, _):
            base = wid * TEP + cb * CA
            pltpu.sync_copy(srcp_hbm.at[pl.ds(base, CA)], src_idx)
            pltpu.sync_copy(dst_hbm.at[pl.ds(base, CA)], dst_idx)
            ga = pltpu.async_copy(hsd_hbm.at[src_idx], rows_s, sem0)
            gb = pltpu.async_copy(hsd_hbm.at[dst_idx], rows_d, sem1)
            ga.wait()
            gb.wait()

            def ebody(e, _):
                e2 = 2 * e
                a0 = row16(rows_s, e2)
                b0 = row16(rows_d, e2)
                a1 = row16(rows_s, e2 + 1)
                b1 = row16(rows_d, e2 + 1)
                sh = jnp.bitwise_and(iot + 8, 15)
                v0 = a0 + _vgather(b0, sh)
                v1 = a1 + _vgather(b1, sh)
                m = jnp.where(iot < 8, v0, _vgather(v1, sh))
                m = jnp.where(m >= 0, m, 0.2 * m)
                exv = jnp.exp(m)
                g0 = base + e2
                sel = jnp.where(iot < 8, g0 < E2, g0 + 1 < E2)
                exv = jnp.where(sel, exv, 0.0)
                rows16 = e2 + jnp.where(iot < 8, 0, 1)
                plsc.store_scatter(ex_buf, [rows16, jnp.bitwise_and(iot, 7)], exv)
                return 0
            lax.fori_loop(0, CA // 2, ebody, 0)

            pltpu.sync_copy(ex_buf, den_sp.at[dst_idx], add=True)
            pltpu.sync_copy(ex_buf, ex_hbm.at[pl.ds(base, CA), :])
            return 0
        lax.fori_loop(0, TEP // CA, chunk_body, 0)

    plsc.subcore_barrier()
    pltpu.sync_copy(den_sp0.at[pl.ds(s * NSL, NSL)],
                    den_hbm.at[0, c, pl.ds(s * NSL, NSL), :])
    pltpu.sync_copy(den_sp1.at[pl.ds(s * NSL, NSL)],
                    den_hbm.at[1, c, pl.ds(s * NSL, NSL), :])


def _pass_a(srcp, dstp0, dstp1, hsd0, hsd1, z8):
    return pl.kernel(
        _pass_a_body,
        out_type=[
            jax.ShapeDtypeStruct((EPAD, 16), jnp.float32),
            jax.ShapeDtypeStruct((EPAD, 16), jnp.float32),
            jax.ShapeDtypeStruct((HOPS, NC, NP, 16), jnp.float32),
        ],
        mesh=_MESH,
        compiler_params=pltpu.CompilerParams(
            needs_layout_passes=False, use_tc_tiling_on_sc=False),
        scratch_types=[
            pltpu.VMEM((CA,), jnp.int32),
            pltpu.VMEM((CA,), jnp.int32),
            pltpu.VMEM((CA, 16), jnp.float32),
            pltpu.VMEM((CA, 16), jnp.float32),
            pltpu.VMEM((CA, 16), jnp.float32),
            pltpu.VMEM_SHARED((NP, 16), jnp.float32),
            pltpu.VMEM_SHARED((NP, 16), jnp.float32),
            pltpu.SemaphoreType.DMA,
            pltpu.SemaphoreType.DMA,
        ],
    )(srcp, dstp0, dstp1, hsd0, hsd1, z8)


CB = 344
TEP2 = EPAD // NS


def _pass_b_body(srcp_hbm, dstp_hbm, ex_hbm, dena_hbm, denb_hbm,
                 hwa_hbm, hwb_hbm, z64_hbm,
                 out_hbm,
                 src_idx, dst_idx, ex_v, d0_v, d1_v, msg_v,
                 out_sp, sem0, sem1, sem2, sem3):
    c = lax.axis_index("c")
    s = lax.axis_index("s")
    iot = _IOTA16()

    pltpu.sync_copy(z64_hbm.at[pl.ds(s * NSL, NSL)], out_sp.at[pl.ds(s * NSL, NSL)])
    plsc.subcore_barrier()

    hoff = c * 4
    cols4 = hoff + jnp.bitwise_and(iot, 3)
    lane_e = lax.shift_right_logical(iot, 2)

    def chunk_body(cb, _):
        base = s * TEP2 + cb * CB
        pltpu.sync_copy(srcp_hbm.at[pl.ds(base, CB)], src_idx)
        pltpu.sync_copy(dstp_hbm.at[pl.ds(base, CB)], dst_idx)
        g0 = pltpu.async_copy(ex_hbm.at[pl.ds(base, CB), :], ex_v, sem0)
        g1 = pltpu.async_copy(dena_hbm.at[dst_idx], d0_v, sem1)
        g2 = pltpu.async_copy(denb_hbm.at[dst_idx], d1_v, sem2)

        @pl.when(c == 0)
        def _():
            pltpu.async_copy(hwa_hbm.at[src_idx], msg_v, sem3).wait()

        @pl.when(c == 1)
        def _():
            pltpu.async_copy(hwb_hbm.at[src_idx], msg_v, sem3).wait()

        g0.wait()
        g1.wait()
        g2.wait()

        def ebody(e, _):
            e4 = 4 * e
            rows16 = e4 + lane_e
            exv = plsc.load_gather(ex_v, [rows16, cols4])
            dn0 = plsc.load_gather(d0_v, [rows16, cols4])
            dn1 = plsc.load_gather(d1_v, [rows16, cols4])
            alpha = exv / (dn0 + dn1 + 1e-16)
            for q in range(4):
                for h in range(4):
                    a = _vgather(alpha, jnp.full((16,), 4 * q + h, jnp.int32))
                    r16 = jnp.full((16,), e4 + q, jnp.int32)
                    c16 = h * 16 + iot
                    r = plsc.load_gather(msg_v, [r16, c16])
                    plsc.store_scatter(msg_v, [r16, c16], r * a)
            return 0
        lax.fori_loop(0, CB // 4, ebody, 0)

        pltpu.sync_copy(msg_v, out_sp.at[dst_idx], add=True)
        return 0
    lax.fori_loop(0, TEP2 // CB, chunk_body, 0)

    plsc.subcore_barrier()
    pltpu.sync_copy(out_sp.at[pl.ds(s * NSL, NSL)],
                    out_hbm.at[c, pl.ds(s * NSL, NSL), :])


def _pass_b(srcp, dstpk, exk, dena, denb, hwa, hwb, z64):
    return pl.kernel(
        _pass_b_body,
        out_type=jax.ShapeDtypeStruct((NC, NP, 64), jnp.float32),
        mesh=_MESH,
        compiler_params=pltpu.CompilerParams(
            needs_layout_passes=False, use_tc_tiling_on_sc=False),
        scratch_types=[
            pltpu.VMEM((CB,), jnp.int32),
            pltpu.VMEM((CB,), jnp.int32),
            pltpu.VMEM((CB, 16), jnp.float32),
            pltpu.VMEM((CB, 16), jnp.float32),
            pltpu.VMEM((CB, 16), jnp.float32),
            pltpu.VMEM((CB, 64), jnp.float32),
            pltpu.VMEM_SHARED((NP, 64), jnp.float32),
            pltpu.SemaphoreType.DMA,
            pltpu.SemaphoreType.DMA,
            pltpu.SemaphoreType.DMA,
            pltpu.SemaphoreType.DMA,
        ],
    )(srcp, dstpk, exk, dena, denb, hwa, hwb, z64)


_BT = 400


def _proj_body(h_ref, w_ref, asd_ref, hwa_ref, hwb_ref, hsd_ref):
    h = h_ref[...]
    hw = jnp.dot(h, w_ref[0], preferred_element_type=jnp.float32)
    hwa_ref[0] = hw[:, :64]
    hwb_ref[0] = hw[:, 64:]
    h3 = hw.reshape(_BT, HEADS, OPH)
    hs = (h3 * asd_ref[0, 0]).sum(-1)
    hd = (h3 * asd_ref[0, 1]).sum(-1)
    hsd_ref[0] = jnp.concatenate([hs, hd], axis=1)


def _proj(h, Wl, asl, adl):
    asd = jnp.stack([asl, adl], axis=1)
    return pl.pallas_call(
        _proj_body,
        grid=(HOPS, N // _BT),
        in_specs=[
            pl.BlockSpec((_BT, 128), lambda k, i: (i, 0)),
            pl.BlockSpec((1, 128, 128), lambda k, i: (k, 0, 0)),
            pl.BlockSpec((1, 2, HEADS, OPH), lambda k, i: (k, 0, 0, 0)),
        ],
        out_specs=[
            pl.BlockSpec((1, _BT, 64), lambda k, i: (k, i, 0)),
            pl.BlockSpec((1, _BT, 64), lambda k, i: (k, i, 0)),
            pl.BlockSpec((1, _BT, 16), lambda k, i: (k, i, 0)),
        ],
        out_shape=[
            jax.ShapeDtypeStruct((HOPS, N, 64), jnp.float32),
            jax.ShapeDtypeStruct((HOPS, N, 64), jnp.float32),
            jax.ShapeDtypeStruct((HOPS, N, 16), jnp.float32),
        ],
    )(h, Wl, asd)


def _dec_body(g0a_ref, g0b_ref, g1a_ref, g1b_ref, gb_ref, dw_ref, db_ref,
              lg_ref, lb_ref, res_ref, o_ref):
    x0 = jnp.concatenate([g0a_ref[0], g0b_ref[0]], axis=1) + gb_ref[0]
    x1 = jnp.concatenate([g1a_ref[0], g1b_ref[0]], axis=1) + gb_ref[1]
    x0 = jnp.dot(x0, dw_ref[0], preferred_element_type=jnp.float32) + db_ref[0]
    x1 = jnp.dot(x1, dw_ref[1], preferred_element_type=jnp.float32) + db_ref[1]
    acc = DECAY[0] * _lrelu(x0, 0.01) + DECAY[1] * _lrelu(x1, 0.01)
    mu = acc.mean(axis=-1, keepdims=True)
    var = ((acc - mu) ** 2).mean(axis=-1, keepdims=True)
    xl = (acc - mu) / jnp.sqrt(var + 1e-5) * lg_ref[...] + lb_ref[...]
    o_ref[...] = xl + res_ref[...]


def _decode(g0, g1, gbl, dwl, dbl, lgl, lbl, res):
    blk64 = lambda c: pl.BlockSpec((1, _BT, 64), lambda i, c=c: (c, i, 0))
    return pl.pallas_call(
        _dec_body,
        grid=(N // _BT,),
        in_specs=[
            blk64(0), blk64(1), blk64(0), blk64(1),
            pl.BlockSpec((2, 128), lambda i: (0, 0)),
            pl.BlockSpec((2, 128, 128), lambda i: (0, 0, 0)),
            pl.BlockSpec((2, 128), lambda i: (0, 0)),
            pl.BlockSpec((1, 128), lambda i: (0, 0)),
            pl.BlockSpec((1, 128), lambda i: (0, 0)),
            pl.BlockSpec((_BT, 128), lambda i: (i, 0)),
        ],
        out_specs=pl.BlockSpec((_BT, 128), lambda i: (i, 0)),
        out_shape=jax.ShapeDtypeStruct((N, 128), jnp.float32),
    )(g0, g0, g1, g1, gbl, dwl, dbl, lgl.reshape(1, 128), lbl.reshape(1, 128), res)


def kernel(x, edge_index, edge_type, genre, genre_mask, W1, b1, gat_W, att_src, att_dst, gat_b, dec_W, dec_b, ln_g, ln_b):
    src0, dst0 = edge_index[0], edge_index[1]
    nbr_p = _nbr_partials(src0, dst0)
    dst1 = _dst1_compute(dst0, nbr_p)
    loop = jnp.arange(N, dtype=edge_index.dtype)
    pad = jnp.zeros((EPAD - E2,), jnp.int32)
    srcp = jnp.concatenate([src0, loop, pad])
    dstp = [jnp.concatenate([dst0, loop, pad]), jnp.concatenate([dst1, loop, pad])]
    z8 = jnp.zeros((NP, 16), jnp.float32)
    z64 = jnp.zeros((NP, 64), jnp.float32)

    h = _stage0(x, W1, b1)
    residual = h
    for l in range(LAYERS):
        hwa, hwb, hsd = _proj(h, gat_W[l], att_src[l], att_dst[l])
        ex0, ex1, den = _pass_a(srcp, dstp[0], dstp[1], hsd[0], hsd[1], z8)
        exs = [ex0, ex1]
        g = [
            _pass_b(srcp, dstp[k], exs[k], den[k, 0], den[k, 1],
                    hwa[k], hwb[k], z64)
            for k in range(HOPS)
        ]
        h = _decode(g[0], g[1], gat_b[l], dec_W[l], dec_b[l],
                    ln_g[l], ln_b[l], residual)
        residual = h
    return h


# speedup vs baseline: 1.2195x; 1.2195x over previous
import functools

import jax
import jax.numpy as jnp
import numpy as np
from jax import lax
from jax.experimental import pallas as pl
from jax.experimental.pallas import tpu as pltpu
from jax.experimental.pallas import tpu_sc as plsc

N = 10000
E = 320000
HEADS = 8
OPH = 16
LAYERS = 2
HOPS = 2
DECAY = [float(np.exp(-0.5 * k)) for k in range(HOPS)]

NC, NS, LANES = 2, 16, 16
NW = NC * NS
NP = 10240
NSL = NP // NS
TE = E // NW

_MESH = plsc.VectorSubcoreMesh(
    core_axis_name="c", subcore_axis_name="s", num_cores=NC, num_subcores=NS)

_IOTA16 = lambda: lax.iota(jnp.int32, 16)


def _vgather(v, idx):
    return lax.gather(
        v, idx[:, None],
        lax.GatherDimensionNumbers(
            offset_dims=(), collapsed_slice_dims=(0,), start_index_map=(0,)),
        (1,), mode=lax.GatherScatterMode.PROMISE_IN_BOUNDS)


def _nbr_body(src_hbm, dst_hbm, out_hbm, src_v, dst_v, tbl_v, blk_v, acc_v, shr):
    c = lax.axis_index("c")
    s = lax.axis_index("s")
    wid = s * NC + c
    pltpu.sync_copy(src_hbm.at[pl.ds(wid * TE, TE)], src_v)
    pltpu.sync_copy(dst_hbm.at[pl.ds(wid * TE, TE)], dst_v)

    def zbody(i, _):
        tbl_v[pl.ds(i * 16, 16)] = jnp.zeros((16,), jnp.int32)
        return 0
    lax.fori_loop(0, NP // 16, zbody, 0)

    iot = _IOTA16()

    def ebody(i, _):
        sv = src_v[pl.ds(i * 16, 16)]
        dv = dst_v[pl.ds(i * 16, 16)]
        ks, _ = plsc.sort_key_val(sv * 16384 + dv, dv)
        ss = lax.shift_right_logical(ks, 14)
        dd = jnp.bitwise_and(ks, 16383)
        nxt = _vgather(ss, jnp.minimum(iot + 1, 15))
        is_end = jnp.logical_or(ss != nxt, iot == 15)
        old = plsc.load_gather(tbl_v, [ss], mask=is_end)
        plsc.store_scatter(tbl_v, [ss], jnp.maximum(old, dd), mask=is_end)
        return 0
    lax.fori_loop(0, TE // 16, ebody, 0)

    pltpu.sync_copy(tbl_v, shr.at[s])
    plsc.subcore_barrier()
    for r in range(NS):
        pltpu.sync_copy(shr.at[r, pl.ds(s * NSL, NSL)],
                        blk_v.at[pl.ds(r * NSL, NSL)])

    def cbody(j, _):
        m = blk_v[pl.ds(j * 16, 16)]
        for r in range(1, NS):
            m = jnp.maximum(m, blk_v[pl.ds(r * NSL + j * 16, 16)])
        acc_v[pl.ds(j * 16, 16)] = m
        return 0
    lax.fori_loop(0, NSL // 16, cbody, 0)
    pltpu.sync_copy(acc_v, out_hbm.at[c, pl.ds(s * NSL, NSL)])


@jax.jit
def _nbr_partials(src, dst):
    return pl.kernel(
        _nbr_body,
        out_type=jax.ShapeDtypeStruct((NC, NP), jnp.int32),
        mesh=_MESH,
        compiler_params=pltpu.CompilerParams(needs_layout_passes=False),
        scratch_types=[
            pltpu.VMEM((TE,), jnp.int32),
            pltpu.VMEM((TE,), jnp.int32),
            pltpu.VMEM((NP,), jnp.int32),
            pltpu.VMEM((NS * NSL,), jnp.int32),
            pltpu.VMEM((NSL,), jnp.int32),
            pltpu.VMEM_SHARED((NS, NP), jnp.int32),
        ],
    )(src, dst)


def _dst1_body(dst_hbm, nbr_hbm, out_hbm, dst_v, t0_v, t1_v, o_v):
    c = lax.axis_index("c")
    s = lax.axis_index("s")
    wid = s * NC + c
    pltpu.sync_copy(dst_hbm.at[pl.ds(wid * TE, TE)], dst_v)
    pltpu.sync_copy(nbr_hbm.at[0], t0_v)
    pltpu.sync_copy(nbr_hbm.at[1], t1_v)

    def mb(j, _):
        t0_v[pl.ds(j * 16, 16)] = jnp.maximum(
            t0_v[pl.ds(j * 16, 16)], t1_v[pl.ds(j * 16, 16)])
        return 0
    lax.fori_loop(0, NP // 16, mb, 0)

    def eb(i, _):
        dv = dst_v[pl.ds(i * 16, 16)]
        o_v[pl.ds(i * 16, 16)] = plsc.load_gather(t0_v, [dv])
        return 0
    lax.fori_loop(0, TE // 16, eb, 0)
    pltpu.sync_copy(o_v, out_hbm.at[pl.ds(wid * TE, TE)])


@jax.jit
def _dst1_compute(dst, nbr_p):
    return pl.kernel(
        _dst1_body,
        out_type=jax.ShapeDtypeStruct((E,), jnp.int32),
        mesh=_MESH,
        compiler_params=pltpu.CompilerParams(needs_layout_passes=False),
        scratch_types=[
            pltpu.VMEM((TE,), jnp.int32),
            pltpu.VMEM((NP,), jnp.int32),
            pltpu.VMEM((NP,), jnp.int32),
            pltpu.VMEM((TE,), jnp.int32),
        ],
    )(dst, nbr_p)


def _lrelu(v, s):
    return jnp.where(v >= 0, v, s * v)


def _stage0_body(x_ref, w_ref, b_ref, o_ref):
    h = jnp.dot(x_ref[...], w_ref[...], preferred_element_type=jnp.float32) + b_ref[...]
    o_ref[...] = _lrelu(h, 0.01)


def _stage0(x, W1, b1):
    B = 400
    return pl.pallas_call(
        _stage0_body,
        grid=(N // B,),
        in_specs=[
            pl.BlockSpec((B, 128), lambda i: (i, 0)),
            pl.BlockSpec((128, 128), lambda i: (0, 0)),
            pl.BlockSpec((1, 128), lambda i: (0, 0)),
        ],
        out_specs=pl.BlockSpec((B, 128), lambda i: (i, 0)),
        out_shape=jax.ShapeDtypeStruct((N, 128), jnp.float32),
    )(x, W1, b1.reshape(1, 128))


E2 = E + N
EPAD = 330240
TEP = EPAD // NW
CA = 1032


def _pass_a_body(srcp_hbm, dst0_hbm, dst1_hbm, hsd0_hbm, hsd1_hbm, z8_hbm,
                 ex0_hbm, ex1_hbm, den_hbm,
                 src_idx, dst_idx, rows_s, rows_d, ex_buf,
                 den_sp0, den_sp1, sem0, sem1):
    c = lax.axis_index("c")
    s = lax.axis_index("s")
    wid = s * NC + c
    iot = _IOTA16()

    pltpu.sync_copy(z8_hbm.at[pl.ds(s * NSL, NSL)], den_sp0.at[pl.ds(s * NSL, NSL)])
    pltpu.sync_copy(z8_hbm.at[pl.ds(s * NSL, NSL)], den_sp1.at[pl.ds(s * NSL, NSL)])
    pltpu.sync_copy(z8_hbm.at[pl.ds(0, CA), :], ex_buf)
    plsc.subcore_barrier()

    def row16(ref, r):
        return plsc.load_gather(ref, [jnp.full((16,), r, jnp.int32), iot])

    for k in range(HOPS):
        dst_hbm = dst0_hbm if k == 0 else dst1_hbm
        hsd_hbm = hsd0_hbm if k == 0 else hsd1_hbm
        ex_hbm = ex0_hbm if k == 0 else ex1_hbm
        den_sp = den_sp0 if k == 0 else den_sp1

        def chunk_body(cb, _):
            base = wid * TEP + cb * CA
            pltpu.sync_copy(srcp_hbm.at[pl.ds(base, CA)], src_idx)
            pltpu.sync_copy(dst_hbm.at[pl.ds(base, CA)], dst_idx)
            ga = pltpu.async_copy(hsd_hbm.at[src_idx], rows_s, sem0)
            gb = pltpu.async_copy(hsd_hbm.at[dst_idx], rows_d, sem1)
            ga.wait()
            gb.wait()

            def ebody(e, _):
                e2 = 2 * e
                a0 = row16(rows_s, e2)
                b0 = row16(rows_d, e2)
                a1 = row16(rows_s, e2 + 1)
                b1 = row16(rows_d, e2 + 1)
                sh = jnp.bitwise_and(iot + 8, 15)
                v0 = a0 + _vgather(b0, sh)
                v1 = a1 + _vgather(b1, sh)
                m = jnp.where(iot < 8, v0, _vgather(v1, sh))
                m = jnp.where(m >= 0, m, 0.2 * m)
                exv = jnp.exp(m)
                g0 = base + e2
                sel = jnp.where(iot < 8, g0 < E2, g0 + 1 < E2)
                exv = jnp.where(sel, exv, 0.0)
                rows16 = e2 + jnp.where(iot < 8, 0, 1)
                plsc.store_scatter(ex_buf, [rows16, jnp.bitwise_and(iot, 7)], exv)
                return 0
            lax.fori_loop(0, CA // 2, ebody, 0)

            pltpu.sync_copy(ex_buf, den_sp.at[dst_idx], add=True)
            pltpu.sync_copy(ex_buf, ex_hbm.at[pl.ds(base, CA), :])
            return 0
        lax.fori_loop(0, TEP // CA, chunk_body, 0)

    plsc.subcore_barrier()
    pltpu.sync_copy(den_sp0.at[pl.ds(s * NSL, NSL)],
                    den_hbm.at[0, c, pl.ds(s * NSL, NSL), :])
    pltpu.sync_copy(den_sp1.at[pl.ds(s * NSL, NSL)],
                    den_hbm.at[1, c, pl.ds(s * NSL, NSL), :])


def _pass_a(srcp, dstp0, dstp1, hsd0, hsd1, z8):
    return pl.kernel(
        _pass_a_body,
        out_type=[
            jax.ShapeDtypeStruct((EPAD, 16), jnp.float32),
            jax.ShapeDtypeStruct((EPAD, 16), jnp.float32),
            jax.ShapeDtypeStruct((HOPS, NC, NP, 16), jnp.float32),
        ],
        mesh=_MESH,
        compiler_params=pltpu.CompilerParams(
            needs_layout_passes=False, use_tc_tiling_on_sc=False),
        scratch_types=[
            pltpu.VMEM((CA,), jnp.int32),
            pltpu.VMEM((CA,), jnp.int32),
            pltpu.VMEM((CA, 16), jnp.float32),
            pltpu.VMEM((CA, 16), jnp.float32),
            pltpu.VMEM((CA, 16), jnp.float32),
            pltpu.VMEM_SHARED((NP, 16), jnp.float32),
            pltpu.VMEM_SHARED((NP, 16), jnp.float32),
            pltpu.SemaphoreType.DMA,
            pltpu.SemaphoreType.DMA,
        ],
    )(srcp, dstp0, dstp1, hsd0, hsd1, z8)


CB = 344
TEP2 = EPAD // NS


NCHB = TEP2 // CB
GCH = 10
NG = NCHB // GCH


def _pass_b_body(srcp_hbm, dstp_hbm, ex_hbm, den_hbm,
                 hwa_hbm, hwb_hbm, z64_hbm,
                 out_hbm,
                 gsrc, gdst, ex_v, d_v, msg_v,
                 out_sp, semE, semD, semM):
    c = lax.axis_index("c")
    s = lax.axis_index("s")
    iot = _IOTA16()

    pltpu.sync_copy(srcp_hbm.at[s, pl.ds(0, GCH)], gsrc[0])
    pltpu.sync_copy(dstp_hbm.at[s, pl.ds(0, GCH)], gdst[0])
    pltpu.sync_copy(z64_hbm.at[pl.ds(s * NSL, NSL)], out_sp.at[pl.ds(s * NSL, NSL)])
    plsc.subcore_barrier()

    hoff = c * 4
    cols4 = hoff + jnp.bitwise_and(iot, 3)
    lane_e = lax.shift_right_logical(iot, 2)

    def issue(i, row, b, gb):
        base = s * TEP2 + i * CB
        pltpu.async_copy(ex_hbm.at[pl.ds(base, CB), :], ex_v[b], semE[b])
        pltpu.async_copy(den_hbm.at[gdst[gb].at[row]], d_v[b], semD[b])

        @pl.when(c == 0)
        def _():
            pltpu.async_copy(hwa_hbm.at[gsrc[gb].at[row]], msg_v[b], semM[b])

        @pl.when(c == 1)
        def _():
            pltpu.async_copy(hwb_hbm.at[gsrc[gb].at[row]], msg_v[b], semM[b])

    def wait(i, row, b, gb):
        base = s * TEP2 + i * CB
        pltpu.make_async_copy(ex_hbm.at[pl.ds(base, CB), :], ex_v[b], semE[b]).wait()
        pltpu.make_async_copy(den_hbm.at[gdst[gb].at[row]], d_v[b], semD[b]).wait()
        pltpu.make_async_copy(hwa_hbm.at[gsrc[gb].at[row]], msg_v[b], semM[b]).wait()

    def compute(row, b, gb):
        def ebody(e, _):
            e4 = 4 * e
            rows16 = e4 + lane_e
            exv = plsc.load_gather(ex_v[b], [rows16, cols4])
            dn = plsc.load_gather(d_v[b], [rows16, cols4])
            alpha = exv / (dn + 1e-16)
            for q in range(4):
                for h in range(4):
                    a = _vgather(alpha, jnp.full((16,), 4 * q + h, jnp.int32))
                    r16 = jnp.full((16,), e4 + q, jnp.int32)
                    c16 = h * 16 + iot
                    r = plsc.load_gather(msg_v[b], [r16, c16])
                    plsc.store_scatter(msg_v[b], [r16, c16], r * a)
            return 0
        lax.fori_loop(0, CB // 4, ebody, 0)
        pltpu.sync_copy(msg_v[b], out_sp.at[gdst[gb].at[row]], add=True)

    issue(0, 0, 0, 0)

    def group(g, gb):
        @pl.when(g + 1 < NG)
        def _():
            pltpu.sync_copy(srcp_hbm.at[s, pl.ds((g + 1) * GCH, GCH)], gsrc[1 - gb])
            pltpu.sync_copy(dstp_hbm.at[s, pl.ds((g + 1) * GCH, GCH)], gdst[1 - gb])

        def pair(j2, _):
            r0 = 2 * j2
            i0 = g * GCH + r0
            wait(i0, r0, 0, gb)
            issue(i0 + 1, r0 + 1, 1, gb)
            compute(r0, 0, gb)
            wait(i0 + 1, r0 + 1, 1, gb)

            @pl.when(j2 < GCH // 2 - 1)
            def _():
                issue(i0 + 2, r0 + 2, 0, gb)
            compute(r0 + 1, 1, gb)
            return 0
        lax.fori_loop(0, GCH // 2, pair, 0)

        @pl.when(g + 1 < NG)
        def _():
            issue((g + 1) * GCH, 0, 0, 1 - gb)

    def gpair(gp, _):
        group(2 * gp, 0)
        group(2 * gp + 1, 1)
        return 0
    lax.fori_loop(0, NG // 2, gpair, 0)

    plsc.subcore_barrier()
    pltpu.sync_copy(out_sp.at[pl.ds(s * NSL, NSL)],
                    out_hbm.at[c, pl.ds(s * NSL, NSL), :])


def _pass_b(srcp_r, dstp_r, exk, dsum, hwa, hwb, z64):
    buf2 = lambda shape, dt: [pltpu.VMEM(shape, dt), pltpu.VMEM(shape, dt)]
    sem2 = lambda: [pltpu.SemaphoreType.DMA, pltpu.SemaphoreType.DMA]
    return pl.kernel(
        _pass_b_body,
        out_type=jax.ShapeDtypeStruct((NC, NP, 64), jnp.float32),
        mesh=_MESH,
        compiler_params=pltpu.CompilerParams(
            needs_layout_passes=False, use_tc_tiling_on_sc=False),
        scratch_types=[
            buf2((GCH, CB), jnp.int32),
            buf2((GCH, CB), jnp.int32),
            buf2((CB, 16), jnp.float32),
            buf2((CB, 16), jnp.float32),
            buf2((CB, 64), jnp.float32),
            pltpu.VMEM_SHARED((NP, 64), jnp.float32),
            sem2(), sem2(), sem2(),
        ],
    )(srcp_r, dstp_r, exk, dsum, hwa, hwb, z64)


def _densum_body(d_ref, o_ref):
    o_ref[0] = d_ref[0, 0] + d_ref[0, 1]


def _densum(den):
    return pl.pallas_call(
        _densum_body,
        grid=(HOPS, NP // 640),
        in_specs=[pl.BlockSpec((1, NC, 640, 16), lambda k, i: (k, 0, i, 0))],
        out_specs=pl.BlockSpec((1, 640, 16), lambda k, i: (k, i, 0)),
        out_shape=jax.ShapeDtypeStruct((HOPS, NP, 16), jnp.float32),
    )(den)


_BT = 400


def _proj_body(h_ref, w_ref, asd_ref, hwa_ref, hwb_ref, hsd_ref):
    h = h_ref[...]
    hw = jnp.dot(h, w_ref[0], preferred_element_type=jnp.float32)
    hwa_ref[0] = hw[:, :64]
    hwb_ref[0] = hw[:, 64:]
    h3 = hw.reshape(_BT, HEADS, OPH)
    hs = (h3 * asd_ref[0, 0]).sum(-1)
    hd = (h3 * asd_ref[0, 1]).sum(-1)
    hsd_ref[0] = jnp.concatenate([hs, hd], axis=1)


def _proj(h, Wl, asl, adl):
    asd = jnp.stack([asl, adl], axis=1)
    return pl.pallas_call(
        _proj_body,
        grid=(HOPS, N // _BT),
        in_specs=[
            pl.BlockSpec((_BT, 128), lambda k, i: (i, 0)),
            pl.BlockSpec((1, 128, 128), lambda k, i: (k, 0, 0)),
            pl.BlockSpec((1, 2, HEADS, OPH), lambda k, i: (k, 0, 0, 0)),
        ],
        out_specs=[
            pl.BlockSpec((1, _BT, 64), lambda k, i: (k, i, 0)),
            pl.BlockSpec((1, _BT, 64), lambda k, i: (k, i, 0)),
            pl.BlockSpec((1, _BT, 16), lambda k, i: (k, i, 0)),
        ],
        out_shape=[
            jax.ShapeDtypeStruct((HOPS, N, 64), jnp.float32),
            jax.ShapeDtypeStruct((HOPS, N, 64), jnp.float32),
            jax.ShapeDtypeStruct((HOPS, N, 16), jnp.float32),
        ],
    )(h, Wl, asd)


def _dec_body(g0a_ref, g0b_ref, g1a_ref, g1b_ref, gb_ref, dw_ref, db_ref,
              lg_ref, lb_ref, res_ref, o_ref):
    x0 = jnp.concatenate([g0a_ref[0], g0b_ref[0]], axis=1) + gb_ref[0]
    x1 = jnp.concatenate([g1a_ref[0], g1b_ref[0]], axis=1) + gb_ref[1]
    x0 = jnp.dot(x0, dw_ref[0], preferred_element_type=jnp.float32) + db_ref[0]
    x1 = jnp.dot(x1, dw_ref[1], preferred_element_type=jnp.float32) + db_ref[1]
    acc = DECAY[0] * _lrelu(x0, 0.01) + DECAY[1] * _lrelu(x1, 0.01)
    mu = acc.mean(axis=-1, keepdims=True)
    var = ((acc - mu) ** 2).mean(axis=-1, keepdims=True)
    xl = (acc - mu) / jnp.sqrt(var + 1e-5) * lg_ref[...] + lb_ref[...]
    o_ref[...] = xl + res_ref[...]


def _decode(g0, g1, gbl, dwl, dbl, lgl, lbl, res):
    blk64 = lambda c: pl.BlockSpec((1, _BT, 64), lambda i, c=c: (c, i, 0))
    return pl.pallas_call(
        _dec_body,
        grid=(N // _BT,),
        in_specs=[
            blk64(0), blk64(1), blk64(0), blk64(1),
            pl.BlockSpec((2, 128), lambda i: (0, 0)),
            pl.BlockSpec((2, 128, 128), lambda i: (0, 0, 0)),
            pl.BlockSpec((2, 128), lambda i: (0, 0)),
            pl.BlockSpec((1, 128), lambda i: (0, 0)),
            pl.BlockSpec((1, 128), lambda i: (0, 0)),
            pl.BlockSpec((_BT, 128), lambda i: (i, 0)),
        ],
        out_specs=pl.BlockSpec((_BT, 128), lambda i: (i, 0)),
        out_shape=jax.ShapeDtypeStruct((N, 128), jnp.float32),
    )(g0, g0, g1, g1, gbl, dwl, dbl, lgl.reshape(1, 128), lbl.reshape(1, 128), res)


def kernel(x, edge_index, edge_type, genre, genre_mask, W1, b1, gat_W, att_src, att_dst, gat_b, dec_W, dec_b, ln_g, ln_b):
    src0, dst0 = edge_index[0], edge_index[1]
    nbr_p = _nbr_partials(src0, dst0)
    dst1 = _dst1_compute(dst0, nbr_p)
    loop = jnp.arange(N, dtype=edge_index.dtype)
    pad = jnp.zeros((EPAD - E2,), jnp.int32)
    srcp = jnp.concatenate([src0, loop, pad])
    dstp = [jnp.concatenate([dst0, loop, pad]), jnp.concatenate([dst1, loop, pad])]
    z8 = jnp.zeros((NP, 16), jnp.float32)
    z64 = jnp.zeros((NP, 64), jnp.float32)
    srcp_r = srcp.reshape(NS, NCHB, CB)
    dstp_r = [d.reshape(NS, NCHB, CB) for d in dstp]

    h = _stage0(x, W1, b1)
    residual = h
    for l in range(LAYERS):
        hwa, hwb, hsd = _proj(h, gat_W[l], att_src[l], att_dst[l])
        ex0, ex1, den = _pass_a(srcp, dstp[0], dstp[1], hsd[0], hsd[1], z8)
        exs = [ex0, ex1]
        dsum = _densum(den)
        g = [
            _pass_b(srcp_r, dstp_r[k], exs[k], dsum[k],
                    hwa[k], hwb[k], z64)
            for k in range(HOPS)
        ]
        h = _decode(g[0], g[1], gat_b[l], dec_W[l], dec_b[l],
                    ln_g[l], ln_b[l], residual)
        residual = h
    return h


# speedup vs baseline: 1.2768x; 1.0469x over previous
import functools

import jax
import jax.numpy as jnp
import numpy as np
from jax import lax
from jax.experimental import pallas as pl
from jax.experimental.pallas import tpu as pltpu
from jax.experimental.pallas import tpu_sc as plsc

N = 10000
E = 320000
HEADS = 8
OPH = 16
LAYERS = 2
HOPS = 2
DECAY = [float(np.exp(-0.5 * k)) for k in range(HOPS)]

NC, NS, LANES = 2, 16, 16
NW = NC * NS
NP = 10240
NSL = NP // NS
TE = E // NW

_MESH = plsc.VectorSubcoreMesh(
    core_axis_name="c", subcore_axis_name="s", num_cores=NC, num_subcores=NS)

_IOTA16 = lambda: lax.iota(jnp.int32, 16)


def _vgather(v, idx):
    return lax.gather(
        v, idx[:, None],
        lax.GatherDimensionNumbers(
            offset_dims=(), collapsed_slice_dims=(0,), start_index_map=(0,)),
        (1,), mode=lax.GatherScatterMode.PROMISE_IN_BOUNDS)


def _nbr_body(src_hbm, dst_hbm, out_hbm, src_v, dst_v, tbl_v, blk_v, acc_v, shr):
    c = lax.axis_index("c")
    s = lax.axis_index("s")
    wid = s * NC + c
    pltpu.sync_copy(src_hbm.at[pl.ds(wid * TE, TE)], src_v)
    pltpu.sync_copy(dst_hbm.at[pl.ds(wid * TE, TE)], dst_v)

    def zbody(i, _):
        tbl_v[pl.ds(i * 16, 16)] = jnp.zeros((16,), jnp.int32)
        return 0
    lax.fori_loop(0, NP // 16, zbody, 0)

    iot = _IOTA16()

    def ebody(i, _):
        sv = src_v[pl.ds(i * 16, 16)]
        dv = dst_v[pl.ds(i * 16, 16)]
        ks, _ = plsc.sort_key_val(sv * 16384 + dv, dv)
        ss = lax.shift_right_logical(ks, 14)
        dd = jnp.bitwise_and(ks, 16383)
        nxt = _vgather(ss, jnp.minimum(iot + 1, 15))
        is_end = jnp.logical_or(ss != nxt, iot == 15)
        old = plsc.load_gather(tbl_v, [ss], mask=is_end)
        plsc.store_scatter(tbl_v, [ss], jnp.maximum(old, dd), mask=is_end)
        return 0
    lax.fori_loop(0, TE // 16, ebody, 0)

    pltpu.sync_copy(tbl_v, shr.at[s])
    plsc.subcore_barrier()
    for r in range(NS):
        pltpu.sync_copy(shr.at[r, pl.ds(s * NSL, NSL)],
                        blk_v.at[pl.ds(r * NSL, NSL)])

    def cbody(j, _):
        m = blk_v[pl.ds(j * 16, 16)]
        for r in range(1, NS):
            m = jnp.maximum(m, blk_v[pl.ds(r * NSL + j * 16, 16)])
        acc_v[pl.ds(j * 16, 16)] = m
        return 0
    lax.fori_loop(0, NSL // 16, cbody, 0)
    pltpu.sync_copy(acc_v, out_hbm.at[c, pl.ds(s * NSL, NSL)])


@jax.jit
def _nbr_partials(src, dst):
    return pl.kernel(
        _nbr_body,
        out_type=jax.ShapeDtypeStruct((NC, NP), jnp.int32),
        mesh=_MESH,
        compiler_params=pltpu.CompilerParams(needs_layout_passes=False),
        scratch_types=[
            pltpu.VMEM((TE,), jnp.int32),
            pltpu.VMEM((TE,), jnp.int32),
            pltpu.VMEM((NP,), jnp.int32),
            pltpu.VMEM((NS * NSL,), jnp.int32),
            pltpu.VMEM((NSL,), jnp.int32),
            pltpu.VMEM_SHARED((NS, NP), jnp.int32),
        ],
    )(src, dst)


def _dst1_body(dst_hbm, nbr_hbm, out_hbm, dst_v, t0_v, t1_v, o_v):
    c = lax.axis_index("c")
    s = lax.axis_index("s")
    wid = s * NC + c
    pltpu.sync_copy(dst_hbm.at[pl.ds(wid * TE, TE)], dst_v)
    pltpu.sync_copy(nbr_hbm.at[0], t0_v)
    pltpu.sync_copy(nbr_hbm.at[1], t1_v)

    def mb(j, _):
        t0_v[pl.ds(j * 16, 16)] = jnp.maximum(
            t0_v[pl.ds(j * 16, 16)], t1_v[pl.ds(j * 16, 16)])
        return 0
    lax.fori_loop(0, NP // 16, mb, 0)

    def eb(i, _):
        dv = dst_v[pl.ds(i * 16, 16)]
        o_v[pl.ds(i * 16, 16)] = plsc.load_gather(t0_v, [dv])
        return 0
    lax.fori_loop(0, TE // 16, eb, 0)
    pltpu.sync_copy(o_v, out_hbm.at[pl.ds(wid * TE, TE)])


@jax.jit
def _dst1_compute(dst, nbr_p):
    return pl.kernel(
        _dst1_body,
        out_type=jax.ShapeDtypeStruct((E,), jnp.int32),
        mesh=_MESH,
        compiler_params=pltpu.CompilerParams(needs_layout_passes=False),
        scratch_types=[
            pltpu.VMEM((TE,), jnp.int32),
            pltpu.VMEM((NP,), jnp.int32),
            pltpu.VMEM((NP,), jnp.int32),
            pltpu.VMEM((TE,), jnp.int32),
        ],
    )(dst, nbr_p)


def _lrelu(v, s):
    return jnp.where(v >= 0, v, s * v)


def _stage0_body(x_ref, w_ref, b_ref, o_ref):
    h = jnp.dot(x_ref[...], w_ref[...], preferred_element_type=jnp.float32) + b_ref[...]
    o_ref[...] = _lrelu(h, 0.01)


def _stage0(x, W1, b1):
    B = 400
    return pl.pallas_call(
        _stage0_body,
        grid=(N // B,),
        in_specs=[
            pl.BlockSpec((B, 128), lambda i: (i, 0)),
            pl.BlockSpec((128, 128), lambda i: (0, 0)),
            pl.BlockSpec((1, 128), lambda i: (0, 0)),
        ],
        out_specs=pl.BlockSpec((B, 128), lambda i: (i, 0)),
        out_shape=jax.ShapeDtypeStruct((N, 128), jnp.float32),
    )(x, W1, b1.reshape(1, 128))


E2 = E + N
EPAD = 330240
TEP = EPAD // NW
CA = 344


NCHA = TEP // CA


def _pass_a_body(srcp_hbm, dst0_hbm, dst1_hbm, hsd0_hbm, hsd1_hbm, z8_hbm,
                 ex0_hbm, ex1_hbm, den_hbm,
                 src2, dst2, rows_s, rows_d, ex_b,
                 den_sp0, den_sp1, semS, semD):
    c = lax.axis_index("c")
    s = lax.axis_index("s")
    wid = s * NC + c
    iot = _IOTA16()

    pltpu.sync_copy(srcp_hbm.at[wid], src2)
    pltpu.sync_copy(z8_hbm.at[pl.ds(s * NSL, NSL)], den_sp0.at[pl.ds(s * NSL, NSL)])
    pltpu.sync_copy(z8_hbm.at[pl.ds(s * NSL, NSL)], den_sp1.at[pl.ds(s * NSL, NSL)])
    pltpu.sync_copy(z8_hbm.at[pl.ds(0, CA), :], ex_b[0])
    pltpu.sync_copy(z8_hbm.at[pl.ds(0, CA), :], ex_b[1])
    plsc.subcore_barrier()

    def row16(ref, r):
        return plsc.load_gather(ref, [jnp.full((16,), r, jnp.int32), iot])

    cols8 = jnp.bitwise_and(iot, 7)
    sh = jnp.bitwise_and(iot + 8, 15)

    for k in range(HOPS):
        dst_hbm = dst0_hbm if k == 0 else dst1_hbm
        hsd_hbm = hsd0_hbm if k == 0 else hsd1_hbm
        ex_hbm = ex0_hbm if k == 0 else ex1_hbm
        den_sp = den_sp0 if k == 0 else den_sp1

        pltpu.sync_copy(dst_hbm.at[wid], dst2)

        def issue(i, b):
            pltpu.async_copy(hsd_hbm.at[src2.at[i]], rows_s[b], semS[b])
            pltpu.async_copy(hsd_hbm.at[dst2.at[i]], rows_d[b], semD[b])

        def wait(i, b):
            pltpu.make_async_copy(hsd_hbm.at[src2.at[i]], rows_s[b], semS[b]).wait()
            pltpu.make_async_copy(hsd_hbm.at[dst2.at[i]], rows_d[b], semD[b]).wait()

        def compute(i, b):
            base = wid * TEP + i * CA

            def ebody(e, _):
                e2 = 2 * e
                a0 = row16(rows_s[b], e2)
                b0 = row16(rows_d[b], e2)
                a1 = row16(rows_s[b], e2 + 1)
                b1 = row16(rows_d[b], e2 + 1)
                v0 = a0 + _vgather(b0, sh)
                v1 = a1 + _vgather(b1, sh)
                m = jnp.where(iot < 8, v0, _vgather(v1, sh))
                m = jnp.where(m >= 0, m, 0.2 * m)
                exv = jnp.exp(m)
                g0 = base + e2
                sel = jnp.where(iot < 8, g0 < E2, g0 + 1 < E2)
                exv = jnp.where(sel, exv, 0.0)
                rows16 = e2 + jnp.where(iot < 8, 0, 1)
                plsc.store_scatter(ex_b[b], [rows16, cols8], exv)
                return 0
            lax.fori_loop(0, CA // 2, ebody, 0)

            pltpu.sync_copy(ex_b[b], den_sp.at[dst2.at[i]], add=True)
            pltpu.sync_copy(ex_b[b], ex_hbm.at[pl.ds(base, CA), :])

        issue(0, 0)

        def pair(j2, _):
            i0 = 2 * j2
            wait(i0, 0)
            issue(i0 + 1, 1)
            compute(i0, 0)
            wait(i0 + 1, 1)

            @pl.when(j2 < NCHA // 2 - 1)
            def _():
                issue(i0 + 2, 0)
            compute(i0 + 1, 1)
            return 0
        lax.fori_loop(0, NCHA // 2, pair, 0)

    plsc.subcore_barrier()
    pltpu.sync_copy(den_sp0.at[pl.ds(s * NSL, NSL)],
                    den_hbm.at[0, c, pl.ds(s * NSL, NSL), :])
    pltpu.sync_copy(den_sp1.at[pl.ds(s * NSL, NSL)],
                    den_hbm.at[1, c, pl.ds(s * NSL, NSL), :])


def _pass_a(srcp_a, dstp_a0, dstp_a1, hsd0, hsd1, z8):
    buf2 = lambda shape, dt: [pltpu.VMEM(shape, dt), pltpu.VMEM(shape, dt)]
    sem2 = lambda: [pltpu.SemaphoreType.DMA, pltpu.SemaphoreType.DMA]
    return pl.kernel(
        _pass_a_body,
        out_type=[
            jax.ShapeDtypeStruct((EPAD, 16), jnp.float32),
            jax.ShapeDtypeStruct((EPAD, 16), jnp.float32),
            jax.ShapeDtypeStruct((HOPS, NC, NP, 16), jnp.float32),
        ],
        mesh=_MESH,
        compiler_params=pltpu.CompilerParams(
            needs_layout_passes=False, use_tc_tiling_on_sc=False),
        scratch_types=[
            pltpu.VMEM((NCHA, CA), jnp.int32),
            pltpu.VMEM((NCHA, CA), jnp.int32),
            buf2((CA, 16), jnp.float32),
            buf2((CA, 16), jnp.float32),
            buf2((CA, 16), jnp.float32),
            pltpu.VMEM_SHARED((NP, 16), jnp.float32),
            pltpu.VMEM_SHARED((NP, 16), jnp.float32),
            sem2(), sem2(),
        ],
    )(srcp_a, dstp_a0, dstp_a1, hsd0, hsd1, z8)


CB = 344
TEP2 = EPAD // NS


NCHB = TEP2 // CB
GCH = 10
NG = NCHB // GCH


def _pass_b_body(srcp_hbm, dstp_hbm, ex_hbm, den_hbm,
                 hwa_hbm, hwb_hbm, z64_hbm,
                 out_hbm,
                 gsrc, gdst, ex_v, d_v, msg_v,
                 out_sp, semE, semD, semM):
    c = lax.axis_index("c")
    s = lax.axis_index("s")
    iot = _IOTA16()

    pltpu.sync_copy(srcp_hbm.at[s, pl.ds(0, GCH)], gsrc[0])
    pltpu.sync_copy(dstp_hbm.at[s, pl.ds(0, GCH)], gdst[0])
    pltpu.sync_copy(z64_hbm.at[pl.ds(s * NSL, NSL)], out_sp.at[pl.ds(s * NSL, NSL)])
    plsc.subcore_barrier()

    hoff = c * 4
    cols4 = hoff + jnp.bitwise_and(iot, 3)
    lane_e = lax.shift_right_logical(iot, 2)

    def issue(i, row, b, gb):
        base = s * TEP2 + i * CB
        pltpu.async_copy(ex_hbm.at[pl.ds(base, CB), :], ex_v[b], semE[b])
        pltpu.async_copy(den_hbm.at[gdst[gb].at[row]], d_v[b], semD[b])

        @pl.when(c == 0)
        def _():
            pltpu.async_copy(hwa_hbm.at[gsrc[gb].at[row]], msg_v[b], semM[b])

        @pl.when(c == 1)
        def _():
            pltpu.async_copy(hwb_hbm.at[gsrc[gb].at[row]], msg_v[b], semM[b])

    def wait(i, row, b, gb):
        base = s * TEP2 + i * CB
        pltpu.make_async_copy(ex_hbm.at[pl.ds(base, CB), :], ex_v[b], semE[b]).wait()
        pltpu.make_async_copy(den_hbm.at[gdst[gb].at[row]], d_v[b], semD[b]).wait()
        pltpu.make_async_copy(hwa_hbm.at[gsrc[gb].at[row]], msg_v[b], semM[b]).wait()

    def compute(row, b, gb):
        def ebody(e, _):
            e4 = 4 * e
            rows16 = e4 + lane_e
            exv = plsc.load_gather(ex_v[b], [rows16, cols4])
            dn = plsc.load_gather(d_v[b], [rows16, cols4])
            alpha = exv / (dn + 1e-16)
            for q in range(4):
                for h in range(4):
                    a = _vgather(alpha, jnp.full((16,), 4 * q + h, jnp.int32))
                    r16 = jnp.full((16,), e4 + q, jnp.int32)
                    c16 = h * 16 + iot
                    r = plsc.load_gather(msg_v[b], [r16, c16])
                    plsc.store_scatter(msg_v[b], [r16, c16], r * a)
            return 0
        lax.fori_loop(0, CB // 4, ebody, 0)
        pltpu.sync_copy(msg_v[b], out_sp.at[gdst[gb].at[row]], add=True)

    issue(0, 0, 0, 0)

    def group(g, gb):
        @pl.when(g + 1 < NG)
        def _():
            pltpu.sync_copy(srcp_hbm.at[s, pl.ds((g + 1) * GCH, GCH)], gsrc[1 - gb])
            pltpu.sync_copy(dstp_hbm.at[s, pl.ds((g + 1) * GCH, GCH)], gdst[1 - gb])

        def pair(j2, _):
            r0 = 2 * j2
            i0 = g * GCH + r0
            wait(i0, r0, 0, gb)
            issue(i0 + 1, r0 + 1, 1, gb)
            compute(r0, 0, gb)
            wait(i0 + 1, r0 + 1, 1, gb)

            @pl.when(j2 < GCH // 2 - 1)
            def _():
                issue(i0 + 2, r0 + 2, 0, gb)
            compute(r0 + 1, 1, gb)
            return 0
        lax.fori_loop(0, GCH // 2, pair, 0)

        @pl.when(g + 1 < NG)
        def _():
            issue((g + 1) * GCH, 0, 0, 1 - gb)

    def gpair(gp, _):
        group(2 * gp, 0)
        group(2 * gp + 1, 1)
        return 0
    lax.fori_loop(0, NG // 2, gpair, 0)

    plsc.subcore_barrier()
    pltpu.sync_copy(out_sp.at[pl.ds(s * NSL, NSL)],
                    out_hbm.at[c, pl.ds(s * NSL, NSL), :])


def _pass_b(srcp_r, dstp_r, exk, dsum, hwa, hwb, z64):
    buf2 = lambda shape, dt: [pltpu.VMEM(shape, dt), pltpu.VMEM(shape, dt)]
    sem2 = lambda: [pltpu.SemaphoreType.DMA, pltpu.SemaphoreType.DMA]
    return pl.kernel(
        _pass_b_body,
        out_type=jax.ShapeDtypeStruct((NC, NP, 64), jnp.float32),
        mesh=_MESH,
        compiler_params=pltpu.CompilerParams(
            needs_layout_passes=False, use_tc_tiling_on_sc=False),
        scratch_types=[
            buf2((GCH, CB), jnp.int32),
            buf2((GCH, CB), jnp.int32),
            buf2((CB, 16), jnp.float32),
            buf2((CB, 16), jnp.float32),
            buf2((CB, 64), jnp.float32),
            pltpu.VMEM_SHARED((NP, 64), jnp.float32),
            sem2(), sem2(), sem2(),
        ],
    )(srcp_r, dstp_r, exk, dsum, hwa, hwb, z64)


def _densum_body(d_ref, o_ref):
    o_ref[0] = d_ref[0, 0] + d_ref[0, 1]


def _densum(den):
    return pl.pallas_call(
        _densum_body,
        grid=(HOPS, NP // 640),
        in_specs=[pl.BlockSpec((1, NC, 640, 16), lambda k, i: (k, 0, i, 0))],
        out_specs=pl.BlockSpec((1, 640, 16), lambda k, i: (k, i, 0)),
        out_shape=jax.ShapeDtypeStruct((HOPS, NP, 16), jnp.float32),
    )(den)


_BT = 400


def _proj_body(h_ref, w_ref, asd_ref, hwa_ref, hwb_ref, hsd_ref):
    h = h_ref[...]
    hw = jnp.dot(h, w_ref[0], preferred_element_type=jnp.float32)
    hwa_ref[0] = hw[:, :64]
    hwb_ref[0] = hw[:, 64:]
    h3 = hw.reshape(_BT, HEADS, OPH)
    hs = (h3 * asd_ref[0, 0]).sum(-1)
    hd = (h3 * asd_ref[0, 1]).sum(-1)
    hsd_ref[0] = jnp.concatenate([hs, hd], axis=1)


def _proj(h, Wl, asl, adl):
    asd = jnp.stack([asl, adl], axis=1)
    return pl.pallas_call(
        _proj_body,
        grid=(HOPS, N // _BT),
        in_specs=[
            pl.BlockSpec((_BT, 128), lambda k, i: (i, 0)),
            pl.BlockSpec((1, 128, 128), lambda k, i: (k, 0, 0)),
            pl.BlockSpec((1, 2, HEADS, OPH), lambda k, i: (k, 0, 0, 0)),
        ],
        out_specs=[
            pl.BlockSpec((1, _BT, 64), lambda k, i: (k, i, 0)),
            pl.BlockSpec((1, _BT, 64), lambda k, i: (k, i, 0)),
            pl.BlockSpec((1, _BT, 16), lambda k, i: (k, i, 0)),
        ],
        out_shape=[
            jax.ShapeDtypeStruct((HOPS, N, 64), jnp.float32),
            jax.ShapeDtypeStruct((HOPS, N, 64), jnp.float32),
            jax.ShapeDtypeStruct((HOPS, N, 16), jnp.float32),
        ],
    )(h, Wl, asd)


def _dec_body(g0a_ref, g0b_ref, g1a_ref, g1b_ref, gb_ref, dw_ref, db_ref,
              lg_ref, lb_ref, res_ref, o_ref):
    x0 = jnp.concatenate([g0a_ref[0], g0b_ref[0]], axis=1) + gb_ref[0]
    x1 = jnp.concatenate([g1a_ref[0], g1b_ref[0]], axis=1) + gb_ref[1]
    x0 = jnp.dot(x0, dw_ref[0], preferred_element_type=jnp.float32) + db_ref[0]
    x1 = jnp.dot(x1, dw_ref[1], preferred_element_type=jnp.float32) + db_ref[1]
    acc = DECAY[0] * _lrelu(x0, 0.01) + DECAY[1] * _lrelu(x1, 0.01)
    mu = acc.mean(axis=-1, keepdims=True)
    var = ((acc - mu) ** 2).mean(axis=-1, keepdims=True)
    xl = (acc - mu) / jnp.sqrt(var + 1e-5) * lg_ref[...] + lb_ref[...]
    o_ref[...] = xl + res_ref[...]


def _decode(g0, g1, gbl, dwl, dbl, lgl, lbl, res):
    blk64 = lambda c: pl.BlockSpec((1, _BT, 64), lambda i, c=c: (c, i, 0))
    return pl.pallas_call(
        _dec_body,
        grid=(N // _BT,),
        in_specs=[
            blk64(0), blk64(1), blk64(0), blk64(1),
            pl.BlockSpec((2, 128), lambda i: (0, 0)),
            pl.BlockSpec((2, 128, 128), lambda i: (0, 0, 0)),
            pl.BlockSpec((2, 128), lambda i: (0, 0)),
            pl.BlockSpec((1, 128), lambda i: (0, 0)),
            pl.BlockSpec((1, 128), lambda i: (0, 0)),
            pl.BlockSpec((_BT, 128), lambda i: (i, 0)),
        ],
        out_specs=pl.BlockSpec((_BT, 128), lambda i: (i, 0)),
        out_shape=jax.ShapeDtypeStruct((N, 128), jnp.float32),
    )(g0, g0, g1, g1, gbl, dwl, dbl, lgl.reshape(1, 128), lbl.reshape(1, 128), res)


def kernel(x, edge_index, edge_type, genre, genre_mask, W1, b1, gat_W, att_src, att_dst, gat_b, dec_W, dec_b, ln_g, ln_b):
    src0, dst0 = edge_index[0], edge_index[1]
    nbr_p = _nbr_partials(src0, dst0)
    dst1 = _dst1_compute(dst0, nbr_p)
    loop = jnp.arange(N, dtype=edge_index.dtype)
    pad = jnp.zeros((EPAD - E2,), jnp.int32)
    srcp = jnp.concatenate([src0, loop, pad])
    dstp = [jnp.concatenate([dst0, loop, pad]), jnp.concatenate([dst1, loop, pad])]
    z8 = jnp.zeros((NP, 16), jnp.float32)
    z64 = jnp.zeros((NP, 64), jnp.float32)
    srcp_r = srcp.reshape(NS, NCHB, CB)
    dstp_r = [d.reshape(NS, NCHB, CB) for d in dstp]
    srcp_a = srcp.reshape(NW, NCHA, CA)
    dstp_a = [d.reshape(NW, NCHA, CA) for d in dstp]

    h = _stage0(x, W1, b1)
    residual = h
    for l in range(LAYERS):
        hwa, hwb, hsd = _proj(h, gat_W[l], att_src[l], att_dst[l])
        ex0, ex1, den = _pass_a(srcp_a, dstp_a[0], dstp_a[1], hsd[0], hsd[1], z8)
        exs = [ex0, ex1]
        dsum = _densum(den)
        g = [
            _pass_b(srcp_r, dstp_r[k], exs[k], dsum[k],
                    hwa[k], hwb[k], z64)
            for k in range(HOPS)
        ]
        h = _decode(g[0], g[1], gat_b[l], dec_W[l], dec_b[l],
                    ln_g[l], ln_b[l], residual)
        residual = h
    return h


# speedup vs baseline: 2.3006x; 1.8019x over previous
import functools

import jax
import jax.numpy as jnp
import numpy as np
from jax import lax
from jax.experimental import pallas as pl
from jax.experimental.pallas import tpu as pltpu
from jax.experimental.pallas import tpu_sc as plsc

N = 10000
E = 320000
HEADS = 8
OPH = 16
LAYERS = 2
HOPS = 2
DECAY = [float(np.exp(-0.5 * k)) for k in range(HOPS)]

NC, NS, LANES = 2, 16, 16
NW = NC * NS
NP = 10240
NSL = NP // NS
TE = E // NW

_MESH = plsc.VectorSubcoreMesh(
    core_axis_name="c", subcore_axis_name="s", num_cores=NC, num_subcores=NS)

_IOTA16 = lambda: lax.iota(jnp.int32, 16)


def _vgather(v, idx):
    return lax.gather(
        v, idx[:, None],
        lax.GatherDimensionNumbers(
            offset_dims=(), collapsed_slice_dims=(0,), start_index_map=(0,)),
        (1,), mode=lax.GatherScatterMode.PROMISE_IN_BOUNDS)


def _nbr_body(src_hbm, dst_hbm, out_hbm, src_v, dst_v, tbl_v, blk_v, acc_v, shr):
    c = lax.axis_index("c")
    s = lax.axis_index("s")
    wid = s * NC + c
    pltpu.sync_copy(src_hbm.at[pl.ds(wid * TE, TE)], src_v)
    pltpu.sync_copy(dst_hbm.at[pl.ds(wid * TE, TE)], dst_v)

    def zbody(i, _):
        tbl_v[pl.ds(i * 16, 16)] = jnp.zeros((16,), jnp.int32)
        return 0
    lax.fori_loop(0, NP // 16, zbody, 0)

    iot = _IOTA16()

    def ebody(i, _):
        sv = src_v[pl.ds(i * 16, 16)]
        dv = dst_v[pl.ds(i * 16, 16)]
        ks, _ = plsc.sort_key_val(sv * 16384 + dv, dv)
        ss = lax.shift_right_logical(ks, 14)
        dd = jnp.bitwise_and(ks, 16383)
        nxt = _vgather(ss, jnp.minimum(iot + 1, 15))
        is_end = jnp.logical_or(ss != nxt, iot == 15)
        old = plsc.load_gather(tbl_v, [ss], mask=is_end)
        plsc.store_scatter(tbl_v, [ss], jnp.maximum(old, dd), mask=is_end)
        return 0
    lax.fori_loop(0, TE // 16, ebody, 0)

    pltpu.sync_copy(tbl_v, shr.at[s])
    plsc.subcore_barrier()
    for r in range(NS):
        pltpu.sync_copy(shr.at[r, pl.ds(s * NSL, NSL)],
                        blk_v.at[pl.ds(r * NSL, NSL)])

    def cbody(j, _):
        m = blk_v[pl.ds(j * 16, 16)]
        for r in range(1, NS):
            m = jnp.maximum(m, blk_v[pl.ds(r * NSL + j * 16, 16)])
        acc_v[pl.ds(j * 16, 16)] = m
        return 0
    lax.fori_loop(0, NSL // 16, cbody, 0)
    pltpu.sync_copy(acc_v, out_hbm.at[c, pl.ds(s * NSL, NSL)])


@jax.jit
def _nbr_partials(src, dst):
    return pl.kernel(
        _nbr_body,
        out_type=jax.ShapeDtypeStruct((NC, NP), jnp.int32),
        mesh=_MESH,
        compiler_params=pltpu.CompilerParams(needs_layout_passes=False),
        scratch_types=[
            pltpu.VMEM((TE,), jnp.int32),
            pltpu.VMEM((TE,), jnp.int32),
            pltpu.VMEM((NP,), jnp.int32),
            pltpu.VMEM((NS * NSL,), jnp.int32),
            pltpu.VMEM((NSL,), jnp.int32),
            pltpu.VMEM_SHARED((NS, NP), jnp.int32),
        ],
    )(src, dst)


def _dst1_body(dst_hbm, nbr_hbm, out_hbm, dst_v, t0_v, t1_v, o_v):
    c = lax.axis_index("c")
    s = lax.axis_index("s")
    wid = s * NC + c
    pltpu.sync_copy(dst_hbm.at[pl.ds(wid * TE, TE)], dst_v)
    pltpu.sync_copy(nbr_hbm.at[0], t0_v)
    pltpu.sync_copy(nbr_hbm.at[1], t1_v)

    def mb(j, _):
        t0_v[pl.ds(j * 16, 16)] = jnp.maximum(
            t0_v[pl.ds(j * 16, 16)], t1_v[pl.ds(j * 16, 16)])
        return 0
    lax.fori_loop(0, NP // 16, mb, 0)

    def eb(i, _):
        dv = dst_v[pl.ds(i * 16, 16)]
        o_v[pl.ds(i * 16, 16)] = plsc.load_gather(t0_v, [dv])
        return 0
    lax.fori_loop(0, TE // 16, eb, 0)
    pltpu.sync_copy(o_v, out_hbm.at[pl.ds(wid * TE, TE)])


@jax.jit
def _dst1_compute(dst, nbr_p):
    return pl.kernel(
        _dst1_body,
        out_type=jax.ShapeDtypeStruct((E,), jnp.int32),
        mesh=_MESH,
        compiler_params=pltpu.CompilerParams(needs_layout_passes=False),
        scratch_types=[
            pltpu.VMEM((TE,), jnp.int32),
            pltpu.VMEM((NP,), jnp.int32),
            pltpu.VMEM((NP,), jnp.int32),
            pltpu.VMEM((TE,), jnp.int32),
        ],
    )(dst, nbr_p)


def _lrelu(v, s):
    return jnp.where(v >= 0, v, s * v)


def _stage0_body(x_ref, w_ref, b_ref, o_ref):
    h = jnp.dot(x_ref[...], w_ref[...], preferred_element_type=jnp.float32) + b_ref[...]
    o_ref[...] = _lrelu(h, 0.01)


def _stage0(x, W1, b1):
    B = 400
    return pl.pallas_call(
        _stage0_body,
        grid=(N // B,),
        in_specs=[
            pl.BlockSpec((B, 128), lambda i: (i, 0)),
            pl.BlockSpec((128, 128), lambda i: (0, 0)),
            pl.BlockSpec((1, 128), lambda i: (0, 0)),
        ],
        out_specs=pl.BlockSpec((B, 128), lambda i: (i, 0)),
        out_shape=jax.ShapeDtypeStruct((N, 128), jnp.float32),
    )(x, W1, b1.reshape(1, 128))


E2 = E + N
EPAD = 330240
TEP = EPAD // NW
CA = 344


NCHA = TEP // CA


def _pass_a_body(srcp_hbm, dst0_hbm, dst1_hbm, hsd0_hbm, hsd1_hbm, z8_hbm,
                 ex0_hbm, ex1_hbm, den_hbm,
                 src2, dst2, rows_s, rows_d, ex_b,
                 den_sp0, den_sp1, semS, semD):
    c = lax.axis_index("c")
    s = lax.axis_index("s")
    wid = s * NC + c
    iot = _IOTA16()

    pltpu.sync_copy(srcp_hbm.at[wid], src2)
    pltpu.sync_copy(z8_hbm.at[pl.ds(s * NSL, NSL)], den_sp0.at[pl.ds(s * NSL, NSL)])
    pltpu.sync_copy(z8_hbm.at[pl.ds(s * NSL, NSL)], den_sp1.at[pl.ds(s * NSL, NSL)])
    pltpu.sync_copy(z8_hbm.at[pl.ds(0, CA), :], ex_b[0])
    pltpu.sync_copy(z8_hbm.at[pl.ds(0, CA), :], ex_b[1])
    plsc.subcore_barrier()

    def row16(ref, r):
        return plsc.load_gather(ref, [jnp.full((16,), r, jnp.int32), iot])

    cols8 = jnp.bitwise_and(iot, 7)
    sh = jnp.bitwise_and(iot + 8, 15)

    for k in range(HOPS):
        dst_hbm = dst0_hbm if k == 0 else dst1_hbm
        hsd_hbm = hsd0_hbm if k == 0 else hsd1_hbm
        ex_hbm = ex0_hbm if k == 0 else ex1_hbm
        den_sp = den_sp0 if k == 0 else den_sp1

        pltpu.sync_copy(dst_hbm.at[wid], dst2)

        def issue(i, b):
            pltpu.async_copy(hsd_hbm.at[src2.at[i]], rows_s[b], semS[b])
            pltpu.async_copy(hsd_hbm.at[dst2.at[i]], rows_d[b], semD[b])

        def wait(i, b):
            pltpu.make_async_copy(hsd_hbm.at[src2.at[i]], rows_s[b], semS[b]).wait()
            pltpu.make_async_copy(hsd_hbm.at[dst2.at[i]], rows_d[b], semD[b]).wait()

        def compute(i, b):
            base = wid * TEP + i * CA

            def ebody(e, _):
                e2 = 2 * e
                a0 = row16(rows_s[b], e2)
                b0 = row16(rows_d[b], e2)
                a1 = row16(rows_s[b], e2 + 1)
                b1 = row16(rows_d[b], e2 + 1)
                v0 = a0 + _vgather(b0, sh)
                v1 = a1 + _vgather(b1, sh)
                m = jnp.where(iot < 8, v0, _vgather(v1, sh))
                m = jnp.where(m >= 0, m, 0.2 * m)
                exv = jnp.exp(m)
                g0 = base + e2
                sel = jnp.where(iot < 8, g0 < E2, g0 + 1 < E2)
                exv = jnp.where(sel, exv, 0.0)
                rows16 = e2 + jnp.where(iot < 8, 0, 1)
                plsc.store_scatter(ex_b[b], [rows16, cols8], exv)
                return 0
            lax.fori_loop(0, CA // 2, ebody, 0)

            pltpu.sync_copy(ex_b[b], den_sp.at[dst2.at[i]], add=True)
            pltpu.sync_copy(ex_b[b], ex_hbm.at[pl.ds(base, CA), :])

        issue(0, 0)

        def pair(j2, _):
            i0 = 2 * j2
            wait(i0, 0)
            issue(i0 + 1, 1)
            compute(i0, 0)
            wait(i0 + 1, 1)

            @pl.when(j2 < NCHA // 2 - 1)
            def _():
                issue(i0 + 2, 0)
            compute(i0 + 1, 1)
            return 0
        lax.fori_loop(0, NCHA // 2, pair, 0)

    plsc.subcore_barrier()
    pltpu.sync_copy(den_sp0.at[pl.ds(s * NSL, NSL)],
                    den_hbm.at[0, c, pl.ds(s * NSL, NSL), :])
    pltpu.sync_copy(den_sp1.at[pl.ds(s * NSL, NSL)],
                    den_hbm.at[1, c, pl.ds(s * NSL, NSL), :])


def _pass_a(srcp_a, dstp_a0, dstp_a1, hsd0, hsd1, z8):
    buf2 = lambda shape, dt: [pltpu.VMEM(shape, dt), pltpu.VMEM(shape, dt)]
    sem2 = lambda: [pltpu.SemaphoreType.DMA, pltpu.SemaphoreType.DMA]
    return pl.kernel(
        _pass_a_body,
        out_type=[
            jax.ShapeDtypeStruct((EPAD, 16), jnp.float32),
            jax.ShapeDtypeStruct((EPAD, 16), jnp.float32),
            jax.ShapeDtypeStruct((HOPS, NC, NP, 16), jnp.float32),
        ],
        mesh=_MESH,
        compiler_params=pltpu.CompilerParams(
            needs_layout_passes=False, use_tc_tiling_on_sc=False),
        scratch_types=[
            pltpu.VMEM((NCHA, CA), jnp.int32),
            pltpu.VMEM((NCHA, CA), jnp.int32),
            buf2((CA, 16), jnp.float32),
            buf2((CA, 16), jnp.float32),
            buf2((CA, 16), jnp.float32),
            pltpu.VMEM_SHARED((NP, 16), jnp.float32),
            pltpu.VMEM_SHARED((NP, 16), jnp.float32),
            sem2(), sem2(),
        ],
    )(srcp_a, dstp_a0, dstp_a1, hsd0, hsd1, z8)


CB = 344
TEP2 = EPAD // NS


NCHB = TEP2 // CB
GCH = 10
NG = NCHB // GCH


def _pass_b_body(srcp_hbm, dstp_hbm, ex_hbm, den_hbm,
                 hwa_hbm, hwb_hbm, z64_hbm,
                 out_hbm,
                 gsrc, gdst, ex_v, d_v, msg_v,
                 out_sp, semE, semD, semM):
    c = lax.axis_index("c")
    s = lax.axis_index("s")
    iot = _IOTA16()

    pltpu.sync_copy(srcp_hbm.at[s, pl.ds(0, GCH)], gsrc[0])
    pltpu.sync_copy(dstp_hbm.at[s, pl.ds(0, GCH)], gdst[0])
    pltpu.sync_copy(z64_hbm.at[pl.ds(s * NSL, NSL)], out_sp.at[pl.ds(s * NSL, NSL)])
    plsc.subcore_barrier()

    hoff = c * 4
    cols4 = hoff + jnp.bitwise_and(iot, 3)
    lane_e = lax.shift_right_logical(iot, 2)

    def issue(i, row, b, gb):
        base = s * TEP2 + i * CB
        pltpu.async_copy(ex_hbm.at[pl.ds(base, CB), :], ex_v[b], semE[b])
        pltpu.async_copy(den_hbm.at[gdst[gb].at[row]], d_v[b], semD[b])

        @pl.when(c == 0)
        def _():
            pltpu.async_copy(hwa_hbm.at[gsrc[gb].at[row]], msg_v[b], semM[b])

        @pl.when(c == 1)
        def _():
            pltpu.async_copy(hwb_hbm.at[gsrc[gb].at[row]], msg_v[b], semM[b])

    def wait(i, row, b, gb):
        base = s * TEP2 + i * CB
        pltpu.make_async_copy(ex_hbm.at[pl.ds(base, CB), :], ex_v[b], semE[b]).wait()
        pltpu.make_async_copy(den_hbm.at[gdst[gb].at[row]], d_v[b], semD[b]).wait()
        pltpu.make_async_copy(hwa_hbm.at[gsrc[gb].at[row]], msg_v[b], semM[b]).wait()

    def compute(row, b, gb):
        def ebody(e, _):
            e4 = 4 * e
            rows16 = e4 + lane_e
            exv = plsc.load_gather(ex_v[b], [rows16, cols4])
            dn = plsc.load_gather(d_v[b], [rows16, cols4])
            alpha = exv / (dn + 1e-16)
            for q in range(4):
                for h in range(4):
                    a = _vgather(alpha, jnp.full((16,), 4 * q + h, jnp.int32))
                    r = msg_v[b][e4 + q, pl.ds(h * 16, 16)]
                    msg_v[b][e4 + q, pl.ds(h * 16, 16)] = r * a
            return 0
        lax.fori_loop(0, CB // 4, ebody, 0)
        pltpu.sync_copy(msg_v[b], out_sp.at[gdst[gb].at[row]], add=True)

    issue(0, 0, 0, 0)

    def group(g, gb):
        @pl.when(g + 1 < NG)
        def _():
            pltpu.sync_copy(srcp_hbm.at[s, pl.ds((g + 1) * GCH, GCH)], gsrc[1 - gb])
            pltpu.sync_copy(dstp_hbm.at[s, pl.ds((g + 1) * GCH, GCH)], gdst[1 - gb])

        def pair(j2, _):
            r0 = 2 * j2
            i0 = g * GCH + r0
            wait(i0, r0, 0, gb)
            issue(i0 + 1, r0 + 1, 1, gb)
            compute(r0, 0, gb)
            wait(i0 + 1, r0 + 1, 1, gb)

            @pl.when(j2 < GCH // 2 - 1)
            def _():
                issue(i0 + 2, r0 + 2, 0, gb)
            compute(r0 + 1, 1, gb)
            return 0
        lax.fori_loop(0, GCH // 2, pair, 0)

        @pl.when(g + 1 < NG)
        def _():
            issue((g + 1) * GCH, 0, 0, 1 - gb)

    def gpair(gp, _):
        group(2 * gp, 0)
        group(2 * gp + 1, 1)
        return 0
    lax.fori_loop(0, NG // 2, gpair, 0)

    plsc.subcore_barrier()
    pltpu.sync_copy(out_sp.at[pl.ds(s * NSL, NSL)],
                    out_hbm.at[c, pl.ds(s * NSL, NSL), :])


def _pass_b(srcp_r, dstp_r, exk, dsum, hwa, hwb, z64):
    buf2 = lambda shape, dt: [pltpu.VMEM(shape, dt), pltpu.VMEM(shape, dt)]
    sem2 = lambda: [pltpu.SemaphoreType.DMA, pltpu.SemaphoreType.DMA]
    return pl.kernel(
        _pass_b_body,
        out_type=jax.ShapeDtypeStruct((NC, NP, 64), jnp.float32),
        mesh=_MESH,
        compiler_params=pltpu.CompilerParams(
            needs_layout_passes=False, use_tc_tiling_on_sc=False),
        scratch_types=[
            buf2((GCH, CB), jnp.int32),
            buf2((GCH, CB), jnp.int32),
            buf2((CB, 16), jnp.float32),
            buf2((CB, 16), jnp.float32),
            buf2((CB, 64), jnp.float32),
            pltpu.VMEM_SHARED((NP, 64), jnp.float32),
            sem2(), sem2(), sem2(),
        ],
    )(srcp_r, dstp_r, exk, dsum, hwa, hwb, z64)


def _densum_body(d_ref, o_ref):
    o_ref[0] = d_ref[0, 0] + d_ref[0, 1]


def _densum(den):
    return pl.pallas_call(
        _densum_body,
        grid=(HOPS, NP // 640),
        in_specs=[pl.BlockSpec((1, NC, 640, 16), lambda k, i: (k, 0, i, 0))],
        out_specs=pl.BlockSpec((1, 640, 16), lambda k, i: (k, i, 0)),
        out_shape=jax.ShapeDtypeStruct((HOPS, NP, 16), jnp.float32),
    )(den)


_BT = 400


def _proj_body(h_ref, w_ref, asd_ref, hwa_ref, hwb_ref, hsd_ref):
    h = h_ref[...]
    hw = jnp.dot(h, w_ref[0], preferred_element_type=jnp.float32)
    hwa_ref[0] = hw[:, :64]
    hwb_ref[0] = hw[:, 64:]
    h3 = hw.reshape(_BT, HEADS, OPH)
    hs = (h3 * asd_ref[0, 0]).sum(-1)
    hd = (h3 * asd_ref[0, 1]).sum(-1)
    hsd_ref[0] = jnp.concatenate([hs, hd], axis=1)


def _proj(h, Wl, asl, adl):
    asd = jnp.stack([asl, adl], axis=1)
    return pl.pallas_call(
        _proj_body,
        grid=(HOPS, N // _BT),
        in_specs=[
            pl.BlockSpec((_BT, 128), lambda k, i: (i, 0)),
            pl.BlockSpec((1, 128, 128), lambda k, i: (k, 0, 0)),
            pl.BlockSpec((1, 2, HEADS, OPH), lambda k, i: (k, 0, 0, 0)),
        ],
        out_specs=[
            pl.BlockSpec((1, _BT, 64), lambda k, i: (k, i, 0)),
            pl.BlockSpec((1, _BT, 64), lambda k, i: (k, i, 0)),
            pl.BlockSpec((1, _BT, 16), lambda k, i: (k, i, 0)),
        ],
        out_shape=[
            jax.ShapeDtypeStruct((HOPS, N, 64), jnp.float32),
            jax.ShapeDtypeStruct((HOPS, N, 64), jnp.float32),
            jax.ShapeDtypeStruct((HOPS, N, 16), jnp.float32),
        ],
    )(h, Wl, asd)


def _dec_body(g0a_ref, g0b_ref, g1a_ref, g1b_ref, gb_ref, dw_ref, db_ref,
              lg_ref, lb_ref, res_ref, o_ref):
    x0 = jnp.concatenate([g0a_ref[0], g0b_ref[0]], axis=1) + gb_ref[0]
    x1 = jnp.concatenate([g1a_ref[0], g1b_ref[0]], axis=1) + gb_ref[1]
    x0 = jnp.dot(x0, dw_ref[0], preferred_element_type=jnp.float32) + db_ref[0]
    x1 = jnp.dot(x1, dw_ref[1], preferred_element_type=jnp.float32) + db_ref[1]
    acc = DECAY[0] * _lrelu(x0, 0.01) + DECAY[1] * _lrelu(x1, 0.01)
    mu = acc.mean(axis=-1, keepdims=True)
    var = ((acc - mu) ** 2).mean(axis=-1, keepdims=True)
    xl = (acc - mu) / jnp.sqrt(var + 1e-5) * lg_ref[...] + lb_ref[...]
    o_ref[...] = xl + res_ref[...]


def _decode(g0, g1, gbl, dwl, dbl, lgl, lbl, res):
    blk64 = lambda c: pl.BlockSpec((1, _BT, 64), lambda i, c=c: (c, i, 0))
    return pl.pallas_call(
        _dec_body,
        grid=(N // _BT,),
        in_specs=[
            blk64(0), blk64(1), blk64(0), blk64(1),
            pl.BlockSpec((2, 128), lambda i: (0, 0)),
            pl.BlockSpec((2, 128, 128), lambda i: (0, 0, 0)),
            pl.BlockSpec((2, 128), lambda i: (0, 0)),
            pl.BlockSpec((1, 128), lambda i: (0, 0)),
            pl.BlockSpec((1, 128), lambda i: (0, 0)),
            pl.BlockSpec((_BT, 128), lambda i: (i, 0)),
        ],
        out_specs=pl.BlockSpec((_BT, 128), lambda i: (i, 0)),
        out_shape=jax.ShapeDtypeStruct((N, 128), jnp.float32),
    )(g0, g0, g1, g1, gbl, dwl, dbl, lgl.reshape(1, 128), lbl.reshape(1, 128), res)


def kernel(x, edge_index, edge_type, genre, genre_mask, W1, b1, gat_W, att_src, att_dst, gat_b, dec_W, dec_b, ln_g, ln_b):
    src0, dst0 = edge_index[0], edge_index[1]
    nbr_p = _nbr_partials(src0, dst0)
    dst1 = _dst1_compute(dst0, nbr_p)
    loop = jnp.arange(N, dtype=edge_index.dtype)
    pad = jnp.zeros((EPAD - E2,), jnp.int32)
    srcp = jnp.concatenate([src0, loop, pad])
    dstp = [jnp.concatenate([dst0, loop, pad]), jnp.concatenate([dst1, loop, pad])]
    z8 = jnp.zeros((NP, 16), jnp.float32)
    z64 = jnp.zeros((NP, 64), jnp.float32)
    srcp_r = srcp.reshape(NS, NCHB, CB)
    dstp_r = [d.reshape(NS, NCHB, CB) for d in dstp]
    srcp_a = srcp.reshape(NW, NCHA, CA)
    dstp_a = [d.reshape(NW, NCHA, CA) for d in dstp]

    h = _stage0(x, W1, b1)
    residual = h
    for l in range(LAYERS):
        hwa, hwb, hsd = _proj(h, gat_W[l], att_src[l], att_dst[l])
        ex0, ex1, den = _pass_a(srcp_a, dstp_a[0], dstp_a[1], hsd[0], hsd[1], z8)
        exs = [ex0, ex1]
        dsum = _densum(den)
        g = [
            _pass_b(srcp_r, dstp_r[k], exs[k], dsum[k],
                    hwa[k], hwb[k], z64)
            for k in range(HOPS)
        ]
        h = _decode(g[0], g[1], gat_b[l], dec_W[l], dec_b[l],
                    ln_g[l], ln_b[l], residual)
        residual = h
    return h


# speedup vs baseline: 2.5807x; 1.1218x over previous
import functools

import jax
import jax.numpy as jnp
import numpy as np
from jax import lax
from jax.experimental import pallas as pl
from jax.experimental.pallas import tpu as pltpu
from jax.experimental.pallas import tpu_sc as plsc

N = 10000
E = 320000
HEADS = 8
OPH = 16
LAYERS = 2
HOPS = 2
DECAY = [float(np.exp(-0.5 * k)) for k in range(HOPS)]

NC, NS, LANES = 2, 16, 16
NW = NC * NS
NP = 10240
NSL = NP // NS
TE = E // NW

_MESH = plsc.VectorSubcoreMesh(
    core_axis_name="c", subcore_axis_name="s", num_cores=NC, num_subcores=NS)

_IOTA16 = lambda: lax.iota(jnp.int32, 16)


def _vgather(v, idx):
    return lax.gather(
        v, idx[:, None],
        lax.GatherDimensionNumbers(
            offset_dims=(), collapsed_slice_dims=(0,), start_index_map=(0,)),
        (1,), mode=lax.GatherScatterMode.PROMISE_IN_BOUNDS)


def _nbr_body(src_hbm, dst_hbm, out_hbm, src_v, dst_v, tbl_v, blk_v, acc_v, shr):
    c = lax.axis_index("c")
    s = lax.axis_index("s")
    wid = s * NC + c
    pltpu.sync_copy(src_hbm.at[pl.ds(wid * TE, TE)], src_v)
    pltpu.sync_copy(dst_hbm.at[pl.ds(wid * TE, TE)], dst_v)

    def zbody(i, _):
        tbl_v[pl.ds(i * 16, 16)] = jnp.zeros((16,), jnp.int32)
        return 0
    lax.fori_loop(0, NP // 16, zbody, 0)

    iot = _IOTA16()

    def ebody(i, _):
        sv = src_v[pl.ds(i * 16, 16)]
        dv = dst_v[pl.ds(i * 16, 16)]
        ks, _ = plsc.sort_key_val(sv * 16384 + dv, dv)
        ss = lax.shift_right_logical(ks, 14)
        dd = jnp.bitwise_and(ks, 16383)
        nxt = _vgather(ss, jnp.minimum(iot + 1, 15))
        is_end = jnp.logical_or(ss != nxt, iot == 15)
        old = plsc.load_gather(tbl_v, [ss], mask=is_end)
        plsc.store_scatter(tbl_v, [ss], jnp.maximum(old, dd), mask=is_end)
        return 0
    lax.fori_loop(0, TE // 16, ebody, 0)

    pltpu.sync_copy(tbl_v, shr.at[s])
    plsc.subcore_barrier()
    for r in range(NS):
        pltpu.sync_copy(shr.at[r, pl.ds(s * NSL, NSL)],
                        blk_v.at[pl.ds(r * NSL, NSL)])

    def cbody(j, _):
        m = blk_v[pl.ds(j * 16, 16)]
        for r in range(1, NS):
            m = jnp.maximum(m, blk_v[pl.ds(r * NSL + j * 16, 16)])
        acc_v[pl.ds(j * 16, 16)] = m
        return 0
    lax.fori_loop(0, NSL // 16, cbody, 0)
    pltpu.sync_copy(acc_v, out_hbm.at[c, pl.ds(s * NSL, NSL)])


@jax.jit
def _nbr_partials(src, dst):
    return pl.kernel(
        _nbr_body,
        out_type=jax.ShapeDtypeStruct((NC, NP), jnp.int32),
        mesh=_MESH,
        compiler_params=pltpu.CompilerParams(needs_layout_passes=False),
        scratch_types=[
            pltpu.VMEM((TE,), jnp.int32),
            pltpu.VMEM((TE,), jnp.int32),
            pltpu.VMEM((NP,), jnp.int32),
            pltpu.VMEM((NS * NSL,), jnp.int32),
            pltpu.VMEM((NSL,), jnp.int32),
            pltpu.VMEM_SHARED((NS, NP), jnp.int32),
        ],
    )(src, dst)


def _dst1_body(dst_hbm, nbr_hbm, out_hbm, dst_v, t0_v, t1_v, o_v):
    c = lax.axis_index("c")
    s = lax.axis_index("s")
    wid = s * NC + c
    pltpu.sync_copy(dst_hbm.at[pl.ds(wid * TE, TE)], dst_v)
    pltpu.sync_copy(nbr_hbm.at[0], t0_v)
    pltpu.sync_copy(nbr_hbm.at[1], t1_v)

    def mb(j, _):
        t0_v[pl.ds(j * 16, 16)] = jnp.maximum(
            t0_v[pl.ds(j * 16, 16)], t1_v[pl.ds(j * 16, 16)])
        return 0
    lax.fori_loop(0, NP // 16, mb, 0)

    def eb(i, _):
        dv = dst_v[pl.ds(i * 16, 16)]
        o_v[pl.ds(i * 16, 16)] = plsc.load_gather(t0_v, [dv])
        return 0
    lax.fori_loop(0, TE // 16, eb, 0)
    pltpu.sync_copy(o_v, out_hbm.at[pl.ds(wid * TE, TE)])


@jax.jit
def _dst1_compute(dst, nbr_p):
    return pl.kernel(
        _dst1_body,
        out_type=jax.ShapeDtypeStruct((E,), jnp.int32),
        mesh=_MESH,
        compiler_params=pltpu.CompilerParams(needs_layout_passes=False),
        scratch_types=[
            pltpu.VMEM((TE,), jnp.int32),
            pltpu.VMEM((NP,), jnp.int32),
            pltpu.VMEM((NP,), jnp.int32),
            pltpu.VMEM((TE,), jnp.int32),
        ],
    )(dst, nbr_p)


def _lrelu(v, s):
    return jnp.where(v >= 0, v, s * v)


def _stage0_body(x_ref, w_ref, b_ref, o_ref):
    h = jnp.dot(x_ref[...], w_ref[...], preferred_element_type=jnp.float32) + b_ref[...]
    o_ref[...] = _lrelu(h, 0.01)


def _stage0(x, W1, b1):
    B = 400
    return pl.pallas_call(
        _stage0_body,
        grid=(N // B,),
        in_specs=[
            pl.BlockSpec((B, 128), lambda i: (i, 0)),
            pl.BlockSpec((128, 128), lambda i: (0, 0)),
            pl.BlockSpec((1, 128), lambda i: (0, 0)),
        ],
        out_specs=pl.BlockSpec((B, 128), lambda i: (i, 0)),
        out_shape=jax.ShapeDtypeStruct((N, 128), jnp.float32),
    )(x, W1, b1.reshape(1, 128))


E2 = E + N
EPAD = 330240
TEP = EPAD // NW
CA = 344


NCHA = TEP // CA


def _pass_a_body(srcp_hbm, dst0_hbm, dst1_hbm, hsd0_hbm, hsd1_hbm, z8_hbm,
                 ex0_hbm, ex1_hbm, den_hbm,
                 src2, dst2, rows_s, rows_d, ex_b,
                 den_sp0, den_sp1, semS, semD):
    c = lax.axis_index("c")
    s = lax.axis_index("s")
    wid = s * NC + c
    iot = _IOTA16()

    pltpu.sync_copy(srcp_hbm.at[wid], src2)
    pltpu.sync_copy(z8_hbm.at[pl.ds(s * NSL, NSL)], den_sp0.at[pl.ds(s * NSL, NSL)])
    pltpu.sync_copy(z8_hbm.at[pl.ds(s * NSL, NSL)], den_sp1.at[pl.ds(s * NSL, NSL)])
    pltpu.sync_copy(z8_hbm.at[pl.ds(0, CA), :], ex_b[0])
    pltpu.sync_copy(z8_hbm.at[pl.ds(0, CA), :], ex_b[1])
    plsc.subcore_barrier()

    def row16(ref, r):
        return plsc.load_gather(ref, [jnp.full((16,), r, jnp.int32), iot])

    cols8 = jnp.bitwise_and(iot, 7)
    sh = jnp.bitwise_and(iot + 8, 15)

    for k in range(HOPS):
        dst_hbm = dst0_hbm if k == 0 else dst1_hbm
        hsd_hbm = hsd0_hbm if k == 0 else hsd1_hbm
        ex_hbm = ex0_hbm if k == 0 else ex1_hbm
        den_sp = den_sp0 if k == 0 else den_sp1

        pltpu.sync_copy(dst_hbm.at[wid], dst2)

        def issue(i, b):
            pltpu.async_copy(hsd_hbm.at[src2.at[i]], rows_s[b], semS[b])
            pltpu.async_copy(hsd_hbm.at[dst2.at[i]], rows_d[b], semD[b])

        def wait(i, b):
            pltpu.make_async_copy(hsd_hbm.at[src2.at[i]], rows_s[b], semS[b]).wait()
            pltpu.make_async_copy(hsd_hbm.at[dst2.at[i]], rows_d[b], semD[b]).wait()

        def compute(i, b):
            base = wid * TEP + i * CA

            def ebody(e, _):
                e2 = 2 * e
                for q in range(2):
                    av = rows_s[b][e2 + q, :]
                    bv = rows_d[b][e2 + q, :]
                    v = av + _vgather(bv, sh)
                    v = jnp.where(v >= 0, v, 0.2 * v)
                    exv = jnp.exp(v)
                    keep = jnp.logical_and(iot < 8, base + e2 + q < E2)
                    ex_b[b][e2 + q, :] = jnp.where(keep, exv, 0.0)
                return 0
            lax.fori_loop(0, CA // 2, ebody, 0)

            pltpu.sync_copy(ex_b[b], den_sp.at[dst2.at[i]], add=True)
            pltpu.sync_copy(ex_b[b], ex_hbm.at[pl.ds(base, CA), :])

        issue(0, 0)

        def pair(j2, _):
            i0 = 2 * j2
            wait(i0, 0)
            issue(i0 + 1, 1)
            compute(i0, 0)
            wait(i0 + 1, 1)

            @pl.when(j2 < NCHA // 2 - 1)
            def _():
                issue(i0 + 2, 0)
            compute(i0 + 1, 1)
            return 0
        lax.fori_loop(0, NCHA // 2, pair, 0)

    plsc.subcore_barrier()
    pltpu.sync_copy(den_sp0.at[pl.ds(s * NSL, NSL)],
                    den_hbm.at[0, c, pl.ds(s * NSL, NSL), :])
    pltpu.sync_copy(den_sp1.at[pl.ds(s * NSL, NSL)],
                    den_hbm.at[1, c, pl.ds(s * NSL, NSL), :])


def _pass_a(srcp_a, dstp_a0, dstp_a1, hsd0, hsd1, z8):
    buf2 = lambda shape, dt: [pltpu.VMEM(shape, dt), pltpu.VMEM(shape, dt)]
    sem2 = lambda: [pltpu.SemaphoreType.DMA, pltpu.SemaphoreType.DMA]
    return pl.kernel(
        _pass_a_body,
        out_type=[
            jax.ShapeDtypeStruct((EPAD, 16), jnp.float32),
            jax.ShapeDtypeStruct((EPAD, 16), jnp.float32),
            jax.ShapeDtypeStruct((HOPS, NC, NP, 16), jnp.float32),
        ],
        mesh=_MESH,
        compiler_params=pltpu.CompilerParams(
            needs_layout_passes=False, use_tc_tiling_on_sc=False),
        scratch_types=[
            pltpu.VMEM((NCHA, CA), jnp.int32),
            pltpu.VMEM((NCHA, CA), jnp.int32),
            buf2((CA, 16), jnp.float32),
            buf2((CA, 16), jnp.float32),
            buf2((CA, 16), jnp.float32),
            pltpu.VMEM_SHARED((NP, 16), jnp.float32),
            pltpu.VMEM_SHARED((NP, 16), jnp.float32),
            sem2(), sem2(),
        ],
    )(srcp_a, dstp_a0, dstp_a1, hsd0, hsd1, z8)


CB = 344
TEP2 = EPAD // NS


NCHB = TEP2 // CB
GCH = 10
NG = NCHB // GCH


def _pass_b_body(srcp_hbm, dstp_hbm, ex_hbm, den_hbm,
                 hwa_hbm, hwb_hbm, z64_hbm,
                 out_hbm,
                 gsrc, gdst, ex_v, d_v, msg_v,
                 out_sp, semE, semD, semM):
    c = lax.axis_index("c")
    s = lax.axis_index("s")
    iot = _IOTA16()

    pltpu.sync_copy(srcp_hbm.at[s, pl.ds(0, GCH)], gsrc[0])
    pltpu.sync_copy(dstp_hbm.at[s, pl.ds(0, GCH)], gdst[0])
    pltpu.sync_copy(z64_hbm.at[pl.ds(s * NSL, NSL)], out_sp.at[pl.ds(s * NSL, NSL)])
    plsc.subcore_barrier()

    hoff = c * 4
    cols4 = hoff + jnp.bitwise_and(iot, 3)
    lane_e = lax.shift_right_logical(iot, 2)

    def issue(i, row, b, gb):
        base = s * TEP2 + i * CB
        pltpu.async_copy(ex_hbm.at[pl.ds(base, CB), :], ex_v[b], semE[b])
        pltpu.async_copy(den_hbm.at[gdst[gb].at[row]], d_v[b], semD[b])

        @pl.when(c == 0)
        def _():
            pltpu.async_copy(hwa_hbm.at[gsrc[gb].at[row]], msg_v[b], semM[b])

        @pl.when(c == 1)
        def _():
            pltpu.async_copy(hwb_hbm.at[gsrc[gb].at[row]], msg_v[b], semM[b])

    def wait(i, row, b, gb):
        base = s * TEP2 + i * CB
        pltpu.make_async_copy(ex_hbm.at[pl.ds(base, CB), :], ex_v[b], semE[b]).wait()
        pltpu.make_async_copy(den_hbm.at[gdst[gb].at[row]], d_v[b], semD[b]).wait()
        pltpu.make_async_copy(hwa_hbm.at[gsrc[gb].at[row]], msg_v[b], semM[b]).wait()

    def compute(row, b, gb):
        def ebody(e, _):
            e4 = 4 * e
            rows16 = e4 + lane_e
            exv = plsc.load_gather(ex_v[b], [rows16, cols4])
            dn = plsc.load_gather(d_v[b], [rows16, cols4])
            alpha = exv / (dn + 1e-16)
            for q in range(4):
                for h in range(4):
                    a = _vgather(alpha, jnp.full((16,), 4 * q + h, jnp.int32))
                    r = msg_v[b][e4 + q, pl.ds(h * 16, 16)]
                    msg_v[b][e4 + q, pl.ds(h * 16, 16)] = r * a
            return 0
        lax.fori_loop(0, CB // 4, ebody, 0)
        pltpu.sync_copy(msg_v[b], out_sp.at[gdst[gb].at[row]], add=True)

    issue(0, 0, 0, 0)

    def group(g, gb):
        @pl.when(g + 1 < NG)
        def _():
            pltpu.sync_copy(srcp_hbm.at[s, pl.ds((g + 1) * GCH, GCH)], gsrc[1 - gb])
            pltpu.sync_copy(dstp_hbm.at[s, pl.ds((g + 1) * GCH, GCH)], gdst[1 - gb])

        def pair(j2, _):
            r0 = 2 * j2
            i0 = g * GCH + r0
            wait(i0, r0, 0, gb)
            issue(i0 + 1, r0 + 1, 1, gb)
            compute(r0, 0, gb)
            wait(i0 + 1, r0 + 1, 1, gb)

            @pl.when(j2 < GCH // 2 - 1)
            def _():
                issue(i0 + 2, r0 + 2, 0, gb)
            compute(r0 + 1, 1, gb)
            return 0
        lax.fori_loop(0, GCH // 2, pair, 0)

        @pl.when(g + 1 < NG)
        def _():
            issue((g + 1) * GCH, 0, 0, 1 - gb)

    def gpair(gp, _):
        group(2 * gp, 0)
        group(2 * gp + 1, 1)
        return 0
    lax.fori_loop(0, NG // 2, gpair, 0)

    plsc.subcore_barrier()
    pltpu.sync_copy(out_sp.at[pl.ds(s * NSL, NSL)],
                    out_hbm.at[c, pl.ds(s * NSL, NSL), :])


def _pass_b(srcp_r, dstp_r, exk, dsum, hwa, hwb, z64):
    buf2 = lambda shape, dt: [pltpu.VMEM(shape, dt), pltpu.VMEM(shape, dt)]
    sem2 = lambda: [pltpu.SemaphoreType.DMA, pltpu.SemaphoreType.DMA]
    return pl.kernel(
        _pass_b_body,
        out_type=jax.ShapeDtypeStruct((NC, NP, 64), jnp.float32),
        mesh=_MESH,
        compiler_params=pltpu.CompilerParams(
            needs_layout_passes=False, use_tc_tiling_on_sc=False),
        scratch_types=[
            buf2((GCH, CB), jnp.int32),
            buf2((GCH, CB), jnp.int32),
            buf2((CB, 16), jnp.float32),
            buf2((CB, 16), jnp.float32),
            buf2((CB, 64), jnp.float32),
            pltpu.VMEM_SHARED((NP, 64), jnp.float32),
            sem2(), sem2(), sem2(),
        ],
    )(srcp_r, dstp_r, exk, dsum, hwa, hwb, z64)


def _densum_body(d_ref, o_ref):
    o_ref[0] = d_ref[0, 0] + d_ref[0, 1]


def _densum(den):
    return pl.pallas_call(
        _densum_body,
        grid=(HOPS, NP // 640),
        in_specs=[pl.BlockSpec((1, NC, 640, 16), lambda k, i: (k, 0, i, 0))],
        out_specs=pl.BlockSpec((1, 640, 16), lambda k, i: (k, i, 0)),
        out_shape=jax.ShapeDtypeStruct((HOPS, NP, 16), jnp.float32),
    )(den)


_BT = 400


def _proj_body(h_ref, w_ref, asd_ref, hwa_ref, hwb_ref, hsd_ref):
    h = h_ref[...]
    hw = jnp.dot(h, w_ref[0], preferred_element_type=jnp.float32)
    hwa_ref[0] = hw[:, :64]
    hwb_ref[0] = hw[:, 64:]
    h3 = hw.reshape(_BT, HEADS, OPH)
    hs = (h3 * asd_ref[0, 0]).sum(-1)
    hd = (h3 * asd_ref[0, 1]).sum(-1)
    hsd_ref[0] = jnp.concatenate([hs, hd], axis=1)


def _proj(h, Wl, asl, adl):
    asd = jnp.stack([asl, adl], axis=1)
    return pl.pallas_call(
        _proj_body,
        grid=(HOPS, N // _BT),
        in_specs=[
            pl.BlockSpec((_BT, 128), lambda k, i: (i, 0)),
            pl.BlockSpec((1, 128, 128), lambda k, i: (k, 0, 0)),
            pl.BlockSpec((1, 2, HEADS, OPH), lambda k, i: (k, 0, 0, 0)),
        ],
        out_specs=[
            pl.BlockSpec((1, _BT, 64), lambda k, i: (k, i, 0)),
            pl.BlockSpec((1, _BT, 64), lambda k, i: (k, i, 0)),
            pl.BlockSpec((1, _BT, 16), lambda k, i: (k, i, 0)),
        ],
        out_shape=[
            jax.ShapeDtypeStruct((HOPS, N, 64), jnp.float32),
            jax.ShapeDtypeStruct((HOPS, N, 64), jnp.float32),
            jax.ShapeDtypeStruct((HOPS, N, 16), jnp.float32),
        ],
    )(h, Wl, asd)


def _dec_body(g0a_ref, g0b_ref, g1a_ref, g1b_ref, gb_ref, dw_ref, db_ref,
              lg_ref, lb_ref, res_ref, o_ref):
    x0 = jnp.concatenate([g0a_ref[0], g0b_ref[0]], axis=1) + gb_ref[0]
    x1 = jnp.concatenate([g1a_ref[0], g1b_ref[0]], axis=1) + gb_ref[1]
    x0 = jnp.dot(x0, dw_ref[0], preferred_element_type=jnp.float32) + db_ref[0]
    x1 = jnp.dot(x1, dw_ref[1], preferred_element_type=jnp.float32) + db_ref[1]
    acc = DECAY[0] * _lrelu(x0, 0.01) + DECAY[1] * _lrelu(x1, 0.01)
    mu = acc.mean(axis=-1, keepdims=True)
    var = ((acc - mu) ** 2).mean(axis=-1, keepdims=True)
    xl = (acc - mu) / jnp.sqrt(var + 1e-5) * lg_ref[...] + lb_ref[...]
    o_ref[...] = xl + res_ref[...]


def _decode(g0, g1, gbl, dwl, dbl, lgl, lbl, res):
    blk64 = lambda c: pl.BlockSpec((1, _BT, 64), lambda i, c=c: (c, i, 0))
    return pl.pallas_call(
        _dec_body,
        grid=(N // _BT,),
        in_specs=[
            blk64(0), blk64(1), blk64(0), blk64(1),
            pl.BlockSpec((2, 128), lambda i: (0, 0)),
            pl.BlockSpec((2, 128, 128), lambda i: (0, 0, 0)),
            pl.BlockSpec((2, 128), lambda i: (0, 0)),
            pl.BlockSpec((1, 128), lambda i: (0, 0)),
            pl.BlockSpec((1, 128), lambda i: (0, 0)),
            pl.BlockSpec((_BT, 128), lambda i: (i, 0)),
        ],
        out_specs=pl.BlockSpec((_BT, 128), lambda i: (i, 0)),
        out_shape=jax.ShapeDtypeStruct((N, 128), jnp.float32),
    )(g0, g0, g1, g1, gbl, dwl, dbl, lgl.reshape(1, 128), lbl.reshape(1, 128), res)


def kernel(x, edge_index, edge_type, genre, genre_mask, W1, b1, gat_W, att_src, att_dst, gat_b, dec_W, dec_b, ln_g, ln_b):
    src0, dst0 = edge_index[0], edge_index[1]
    nbr_p = _nbr_partials(src0, dst0)
    dst1 = _dst1_compute(dst0, nbr_p)
    loop = jnp.arange(N, dtype=edge_index.dtype)
    pad = jnp.zeros((EPAD - E2,), jnp.int32)
    srcp = jnp.concatenate([src0, loop, pad])
    dstp = [jnp.concatenate([dst0, loop, pad]), jnp.concatenate([dst1, loop, pad])]
    z8 = jnp.zeros((NP, 16), jnp.float32)
    z64 = jnp.zeros((NP, 64), jnp.float32)
    srcp_r = srcp.reshape(NS, NCHB, CB)
    dstp_r = [d.reshape(NS, NCHB, CB) for d in dstp]
    srcp_a = srcp.reshape(NW, NCHA, CA)
    dstp_a = [d.reshape(NW, NCHA, CA) for d in dstp]

    h = _stage0(x, W1, b1)
    residual = h
    for l in range(LAYERS):
        hwa, hwb, hsd = _proj(h, gat_W[l], att_src[l], att_dst[l])
        ex0, ex1, den = _pass_a(srcp_a, dstp_a[0], dstp_a[1], hsd[0], hsd[1], z8)
        exs = [ex0, ex1]
        dsum = _densum(den)
        g = [
            _pass_b(srcp_r, dstp_r[k], exs[k], dsum[k],
                    hwa[k], hwb[k], z64)
            for k in range(HOPS)
        ]
        h = _decode(g[0], g[1], gat_b[l], dec_W[l], dec_b[l],
                    ln_g[l], ln_b[l], residual)
        residual = h
    return h
